# baseline (speedup 1.0000x reference)
import functools

import jax
import jax.numpy as jnp
import numpy as np
from jax.experimental import pallas as pl
from jax.experimental.pallas import tpu as pltpu

LANE = 128


def _round_up(x, m):
    return (x + m - 1) // m * m


@functools.lru_cache(maxsize=2)
def _dft_tables(n0: int, n1: int, ncp: int):
    nc = n1 // 2 + 1
    k = np.arange(ncp)
    n = np.arange(n1)
    ang1 = 2.0 * np.pi * ((np.outer(n, k) % n1) / n1)
    bc = np.cos(ang1)
    bs = np.sin(ang1)
    bc[:, nc:] = 0.0
    bs[:, nc:] = 0.0
    j = np.arange(n0)
    ang0 = 2.0 * np.pi * ((np.outer(j, j) % n0) / n0)
    c0 = np.cos(ang0).astype(np.float32)
    s0 = np.sin(ang0).astype(np.float32)
    return (bc.astype(np.float32), bs.astype(np.float32), c0, s0)


def _rowfft_body(w_ref, bc_ref, bs_ref, fre_ref, fim_ref):
    @pl.when(pl.program_id(1) == 0)
    def _():
        fre_ref[...] = jnp.zeros_like(fre_ref)
        fim_ref[...] = jnp.zeros_like(fim_ref)

    w = w_ref[...]
    fre_ref[...] += jax.lax.dot(w, bc_ref[...],
                                precision=jax.lax.Precision.HIGHEST,
                                preferred_element_type=jnp.float32)
    fim_ref[...] += jax.lax.dot(w, -bs_ref[...],
                                precision=jax.lax.Precision.HIGHEST,
                                preferred_element_type=jnp.float32)


def _rowfft(w, bc, bs, bm=256, bk=256):
    n0, n1 = w.shape
    bm, bk = min(bm, n0), min(bk, n1)
    ncp = bc.shape[1]
    grid = (n0 // bm, n1 // bk)
    return pl.pallas_call(
        _rowfft_body,
        grid=grid,
        in_specs=[
            pl.BlockSpec((bm, bk), lambda i, k: (i, k)),
            pl.BlockSpec((bk, ncp), lambda i, k: (k, 0)),
            pl.BlockSpec((bk, ncp), lambda i, k: (k, 0)),
        ],
        out_specs=[
            pl.BlockSpec((bm, ncp), lambda i, k: (i, 0)),
            pl.BlockSpec((bm, ncp), lambda i, k: (i, 0)),
        ],
        out_shape=[
            jax.ShapeDtypeStruct((n0, ncp), jnp.float32),
            jax.ShapeDtypeStruct((n0, ncp), jnp.float32),
        ],
    )(w, bc, bs)


def _colfft_body(c0_ref, s0_ref, fre_ref, fim_ref,
                 gre_ref, gim_ref, bmax_ref, *, nblk):
    @pl.when(pl.program_id(1) == 0)
    def _():
        gre_ref[...] = jnp.zeros_like(gre_ref)
        gim_ref[...] = jnp.zeros_like(gim_ref)

    c0 = c0_ref[...]
    s0 = s0_ref[...]
    fre = fre_ref[...]
    fim = fim_ref[...]
    hi = jax.lax.Precision.HIGHEST
    gre_ref[...] += (jax.lax.dot(c0, fre, precision=hi,
                                 preferred_element_type=jnp.float32)
                     + jax.lax.dot(s0, fim, precision=hi,
                                   preferred_element_type=jnp.float32))
    gim_ref[...] += (jax.lax.dot(c0, fim, precision=hi,
                                 preferred_element_type=jnp.float32)
                     - jax.lax.dot(s0, fre, precision=hi,
                                   preferred_element_type=jnp.float32))

    @pl.when(pl.program_id(1) == pl.num_programs(1) - 1)
    def _():
        gre = gre_ref[...]
        gim = gim_ref[...]
        mag2 = gre * gre + gim * gim
        for j in range(nblk):
            blk = mag2[:, j * LANE:(j + 1) * LANE]
            bmax_ref[:, j] = jnp.max(blk, axis=1)


def _colfft(c0, s0, fre, fim, bm=256, bk=256):
    n0 = c0.shape[0]
    bm, bk = min(bm, n0), min(bk, n0)
    ncp = fre.shape[1]
    nblk = ncp // LANE
    grid = (n0 // bm, n0 // bk)
    return pl.pallas_call(
        functools.partial(_colfft_body, nblk=nblk),
        grid=grid,
        in_specs=[
            pl.BlockSpec((bm, bk), lambda i, k: (i, k)),
            pl.BlockSpec((bm, bk), lambda i, k: (i, k)),
            pl.BlockSpec((bk, ncp), lambda i, k: (k, 0)),
            pl.BlockSpec((bk, ncp), lambda i, k: (k, 0)),
        ],
        out_specs=[
            pl.BlockSpec((bm, ncp), lambda i, k: (i, 0)),
            pl.BlockSpec((bm, ncp), lambda i, k: (i, 0)),
            pl.BlockSpec((bm, nblk), lambda i, k: (i, 0)),
        ],
        out_shape=[
            jax.ShapeDtypeStruct((n0, ncp), jnp.float32),
            jax.ShapeDtypeStruct((n0, ncp), jnp.float32),
            jax.ShapeDtypeStruct((n0, nblk), jnp.float32),
        ],
    )(c0, s0, fre, fim)


def _basis_body(c0_ref, s0_ref, a_ref, b_ref, r_ref, c_ref,
                u_ref, v_ref, *, n0, n1, nmodes):
    rows = jax.lax.broadcasted_iota(jnp.int32, (c0_ref.shape[0], 1), 0) \
        + pl.program_id(0) * c0_ref.shape[0]
    r = r_ref[0, :]
    c = c_ref[0, :]
    a = a_ref[0, :]
    b = b_ref[0, :]
    onehot_r = (jax.lax.broadcasted_iota(jnp.int32, (c0_ref.shape[1], nmodes), 0)
                == r[None, :]).astype(jnp.float32)
    onehot_c = (jax.lax.broadcasted_iota(jnp.int32, (c0_ref.shape[1], nmodes), 0)
                == c[None, :]).astype(jnp.bfloat16)
    onehot_r = onehot_r.astype(jnp.bfloat16)

    def _gdot(x, oh):
        xh = x.astype(jnp.bfloat16)
        xl = (x - xh.astype(jnp.float32)).astype(jnp.bfloat16)
        return (jax.lax.dot(xh, oh, preferred_element_type=jnp.float32)
                + jax.lax.dot(xl, oh, preferred_element_type=jnp.float32))

    cr = _gdot(c0_ref[...], onehot_r)
    sr = _gdot(s0_ref[...], onehot_r)
    cc = _gdot(c0_ref[...], onehot_c)
    sc = _gdot(s0_ref[...], onehot_c)
    del rows
    p = a[None, :] * cr - b[None, :] * sr
    q = -(a[None, :] * sr + b[None, :] * cr)
    u_ref[...] = jnp.concatenate([p, q], axis=1)
    w = jnp.where((c == 0) | (c == n1 // 2), 1.0, 2.0) / (
        jnp.float32(n0) * jnp.float32(n1))
    v_ref[...] = jnp.concatenate([w[None, :] * cc, w[None, :] * sc], axis=1)


def _basis(c0, s0, a, b, r, c, bm=256):
    n0 = c0.shape[0]
    bm = min(bm, n0)
    nmodes = a.shape[1]
    grid = (n0 // bm,)
    return pl.pallas_call(
        functools.partial(_basis_body, n0=n0, n1=n0, nmodes=nmodes),
        grid=grid,
        in_specs=[
            pl.BlockSpec((bm, n0), lambda i: (i, 0)),
            pl.BlockSpec((bm, n0), lambda i: (i, 0)),
            pl.BlockSpec((1, nmodes), lambda i: (0, 0)),
            pl.BlockSpec((1, nmodes), lambda i: (0, 0)),
            pl.BlockSpec((1, nmodes), lambda i: (0, 0)),
            pl.BlockSpec((1, nmodes), lambda i: (0, 0)),
        ],
        out_specs=[
            pl.BlockSpec((bm, 2 * nmodes), lambda i: (i, 0)),
            pl.BlockSpec((bm, 2 * nmodes), lambda i: (i, 0)),
        ],
        out_shape=[
            jax.ShapeDtypeStruct((n0, 2 * nmodes), jnp.float32),
            jax.ShapeDtypeStruct((n0, 2 * nmodes), jnp.float32),
        ],
    )(c0, s0, a, b, r, c)


def _recon_body(u_ref, v_ref, out_ref):
    out_ref[...] = jax.lax.dot_general(
        u_ref[...], v_ref[...],
        dimension_numbers=(((1,), (1,)), ((), ())),
        precision=jax.lax.Precision.HIGHEST,
        preferred_element_type=jnp.float32)


def _recon(u, v, bm=512, bn=512):
    n0 = u.shape[0]
    bm, bn = min(bm, n0), min(bn, n0)
    kk = u.shape[1]
    grid = (n0 // bm, n0 // bn)
    return pl.pallas_call(
        _recon_body,
        grid=grid,
        in_specs=[
            pl.BlockSpec((bm, kk), lambda i, j: (i, 0)),
            pl.BlockSpec((bn, kk), lambda i, j: (j, 0)),
        ],
        out_specs=pl.BlockSpec((bm, bn), lambda i, j: (i, j)),
        out_shape=jax.ShapeDtypeStruct((n0, n0), jnp.float32),
    )(u, v)


N_KEEP = 64


def kernel(weight_matrix):
    n0, n1 = weight_matrix.shape
    ncp = _round_up(n1 // 2 + 1, LANE)
    bc, bs, c0, s0 = _dft_tables(n0, n1, ncp)
    bc = jnp.asarray(bc)
    bs = jnp.asarray(bs)
    c0 = jnp.asarray(c0)
    s0 = jnp.asarray(s0)

    fre, fim = _rowfft(weight_matrix, bc, bs)
    gre, gim, bmax = _colfft(c0, s0, fre, fim)

    mag2 = gre * gre + gim * gim
    flat = mag2.reshape(-1)
    _, idx = jax.lax.top_k(flat, N_KEEP)
    a = gre.reshape(-1)[idx]
    b = gim.reshape(-1)[idx]
    r = (idx // ncp).astype(jnp.int32)
    c = (idx % ncp).astype(jnp.int32)

    u, v = _basis(c0, s0, a.reshape(1, -1), b.reshape(1, -1),
                  r.reshape(1, -1), c.reshape(1, -1))
    return _recon(u, v)


# speedup vs baseline: 5.0952x; 5.0952x over previous
import functools

import jax
import jax.numpy as jnp
import numpy as np
from jax import lax
from jax.experimental import pallas as pl
from jax.experimental.pallas import tpu as pltpu
from jax.experimental.pallas import tpu_sc as plsc

LANE = 128


def _round_up(x, m):
    return (x + m - 1) // m * m


@functools.lru_cache(maxsize=2)
def _dft_tables(n0: int, n1: int, ncp: int):
    nc = n1 // 2 + 1
    k = np.arange(ncp)
    n = np.arange(n1)
    ang1 = 2.0 * np.pi * ((np.outer(n, k) % n1) / n1)
    bc = np.cos(ang1)
    bs = np.sin(ang1)
    bc[:, nc:] = 0.0
    bs[:, nc:] = 0.0
    j = np.arange(n0)
    ang0 = 2.0 * np.pi * ((np.outer(j, j) % n0) / n0)
    c0 = np.cos(ang0).astype(np.float32)
    s0 = np.sin(ang0).astype(np.float32)
    return (bc.astype(np.float32), bs.astype(np.float32), c0, s0)


def _rowfft_body(w_ref, bc_ref, bs_ref, fre_ref, fim_ref):
    @pl.when(pl.program_id(1) == 0)
    def _():
        fre_ref[...] = jnp.zeros_like(fre_ref)
        fim_ref[...] = jnp.zeros_like(fim_ref)

    w = w_ref[...]
    fre_ref[...] += jax.lax.dot(w, bc_ref[...],
                                precision=jax.lax.Precision.HIGHEST,
                                preferred_element_type=jnp.float32)
    fim_ref[...] += jax.lax.dot(w, -bs_ref[...],
                                precision=jax.lax.Precision.HIGHEST,
                                preferred_element_type=jnp.float32)


def _rowfft(w, bc, bs, bm=256, bk=256):
    n0, n1 = w.shape
    bm, bk = min(bm, n0), min(bk, n1)
    ncp = bc.shape[1]
    grid = (n0 // bm, n1 // bk)
    return pl.pallas_call(
        _rowfft_body,
        grid=grid,
        in_specs=[
            pl.BlockSpec((bm, bk), lambda i, k: (i, k)),
            pl.BlockSpec((bk, ncp), lambda i, k: (k, 0)),
            pl.BlockSpec((bk, ncp), lambda i, k: (k, 0)),
        ],
        out_specs=[
            pl.BlockSpec((bm, ncp), lambda i, k: (i, 0)),
            pl.BlockSpec((bm, ncp), lambda i, k: (i, 0)),
        ],
        out_shape=[
            jax.ShapeDtypeStruct((n0, ncp), jnp.float32),
            jax.ShapeDtypeStruct((n0, ncp), jnp.float32),
        ],
    )(w, bc, bs)


def _colfft_body(c0_ref, s0_ref, fre_ref, fim_ref,
                 gre_ref, gim_ref, bmax_ref, *, nblk):
    @pl.when(pl.program_id(1) == 0)
    def _():
        gre_ref[...] = jnp.zeros_like(gre_ref)
        gim_ref[...] = jnp.zeros_like(gim_ref)

    c0 = c0_ref[...]
    s0 = s0_ref[...]
    fre = fre_ref[...]
    fim = fim_ref[...]
    hi = jax.lax.Precision.HIGHEST
    gre_ref[...] += (jax.lax.dot(c0, fre, precision=hi,
                                 preferred_element_type=jnp.float32)
                     + jax.lax.dot(s0, fim, precision=hi,
                                   preferred_element_type=jnp.float32))
    gim_ref[...] += (jax.lax.dot(c0, fim, precision=hi,
                                 preferred_element_type=jnp.float32)
                     - jax.lax.dot(s0, fre, precision=hi,
                                   preferred_element_type=jnp.float32))

    @pl.when(pl.program_id(1) == pl.num_programs(1) - 1)
    def _():
        gre = gre_ref[...]
        gim = gim_ref[...]
        mag2 = gre * gre + gim * gim
        for j in range(nblk):
            blk = mag2[:, j * LANE:(j + 1) * LANE]
            bmax_ref[:, j] = jnp.max(blk, axis=1)


def _colfft(c0, s0, fre, fim, bm=256, bk=256):
    n0 = c0.shape[0]
    bm, bk = min(bm, n0), min(bk, n0)
    ncp = fre.shape[1]
    nblk = ncp // LANE
    grid = (n0 // bm, n0 // bk)
    return pl.pallas_call(
        functools.partial(_colfft_body, nblk=nblk),
        grid=grid,
        in_specs=[
            pl.BlockSpec((bm, bk), lambda i, k: (i, k)),
            pl.BlockSpec((bm, bk), lambda i, k: (i, k)),
            pl.BlockSpec((bk, ncp), lambda i, k: (k, 0)),
            pl.BlockSpec((bk, ncp), lambda i, k: (k, 0)),
        ],
        out_specs=[
            pl.BlockSpec((bm, ncp), lambda i, k: (i, 0)),
            pl.BlockSpec((bm, ncp), lambda i, k: (i, 0)),
            pl.BlockSpec((bm, nblk), lambda i, k: (i, 0)),
        ],
        out_shape=[
            jax.ShapeDtypeStruct((n0, ncp), jnp.float32),
            jax.ShapeDtypeStruct((n0, ncp), jnp.float32),
            jax.ShapeDtypeStruct((n0, nblk), jnp.float32),
        ],
    )(c0, s0, fre, fim)


def _basis_body(c0_ref, s0_ref, a_ref, b_ref, r_ref, c_ref,
                u_ref, v_ref, *, n0, n1, nmodes):
    rows = jax.lax.broadcasted_iota(jnp.int32, (c0_ref.shape[0], 1), 0) \
        + pl.program_id(0) * c0_ref.shape[0]
    r = r_ref[0, :]
    c = c_ref[0, :]
    a = a_ref[0, :]
    b = b_ref[0, :]
    onehot_r = (jax.lax.broadcasted_iota(jnp.int32, (c0_ref.shape[1], nmodes), 0)
                == r[None, :]).astype(jnp.float32)
    onehot_c = (jax.lax.broadcasted_iota(jnp.int32, (c0_ref.shape[1], nmodes), 0)
                == c[None, :]).astype(jnp.bfloat16)
    onehot_r = onehot_r.astype(jnp.bfloat16)

    def _gdot(x, oh):
        xh = x.astype(jnp.bfloat16)
        xl = (x - xh.astype(jnp.float32)).astype(jnp.bfloat16)
        return (jax.lax.dot(xh, oh, preferred_element_type=jnp.float32)
                + jax.lax.dot(xl, oh, preferred_element_type=jnp.float32))

    cr = _gdot(c0_ref[...], onehot_r)
    sr = _gdot(s0_ref[...], onehot_r)
    cc = _gdot(c0_ref[...], onehot_c)
    sc = _gdot(s0_ref[...], onehot_c)
    del rows
    p = a[None, :] * cr - b[None, :] * sr
    q = -(a[None, :] * sr + b[None, :] * cr)
    u_ref[...] = jnp.concatenate([p, q], axis=1)
    w = jnp.where((c == 0) | (c == n1 // 2), 1.0, 2.0) / (
        jnp.float32(n0) * jnp.float32(n1))
    v_ref[...] = jnp.concatenate([w[None, :] * cc, w[None, :] * sc], axis=1)


def _basis(c0, s0, a, b, r, c, bm=256):
    n0 = c0.shape[0]
    bm = min(bm, n0)
    nmodes = a.shape[1]
    grid = (n0 // bm,)
    return pl.pallas_call(
        functools.partial(_basis_body, n0=n0, n1=n0, nmodes=nmodes),
        grid=grid,
        in_specs=[
            pl.BlockSpec((bm, n0), lambda i: (i, 0)),
            pl.BlockSpec((bm, n0), lambda i: (i, 0)),
            pl.BlockSpec((1, nmodes), lambda i: (0, 0)),
            pl.BlockSpec((1, nmodes), lambda i: (0, 0)),
            pl.BlockSpec((1, nmodes), lambda i: (0, 0)),
            pl.BlockSpec((1, nmodes), lambda i: (0, 0)),
        ],
        out_specs=[
            pl.BlockSpec((bm, 2 * nmodes), lambda i: (i, 0)),
            pl.BlockSpec((bm, 2 * nmodes), lambda i: (i, 0)),
        ],
        out_shape=[
            jax.ShapeDtypeStruct((n0, 2 * nmodes), jnp.float32),
            jax.ShapeDtypeStruct((n0, 2 * nmodes), jnp.float32),
        ],
    )(c0, s0, a, b, r, c)


def _recon_body(u_ref, v_ref, out_ref):
    out_ref[...] = jax.lax.dot_general(
        u_ref[...], v_ref[...],
        dimension_numbers=(((1,), (1,)), ((), ())),
        precision=jax.lax.Precision.HIGHEST,
        preferred_element_type=jnp.float32)


def _recon(u, v, bm=512, bn=512):
    n0 = u.shape[0]
    bm, bn = min(bm, n0), min(bn, n0)
    kk = u.shape[1]
    grid = (n0 // bm, n0 // bn)
    return pl.pallas_call(
        _recon_body,
        grid=grid,
        in_specs=[
            pl.BlockSpec((bm, kk), lambda i, j: (i, 0)),
            pl.BlockSpec((bn, kk), lambda i, j: (j, 0)),
        ],
        out_specs=pl.BlockSpec((bm, bn), lambda i, j: (i, j)),
        out_shape=jax.ShapeDtypeStruct((n0, n0), jnp.float32),
    )(u, v)


def _sc_topk(bmax_flat, gre2, gim2, nblocks, nkeep):
    ns = 16
    pw = nblocks // ns
    nv1 = pw // 16
    rpw = nkeep // ns
    mesh = plsc.VectorSubcoreMesh(core_axis_name="c", subcore_axis_name="s")

    def body(bmax_hbm, gre_hbm, gim_hbm, a_out, b_out, i_out,
             vals1, gidx1, res_v, res_i, merge_v, merge_i, blk_v, blk_i,
             grer, gimr, mvals, mgidx, fin_v, fin_p, outa, outb, outi,
             sh_v, sh_i, sem):
        sid = lax.axis_index("s")
        cid = lax.axis_index("c")
        lane = lax.iota(jnp.int32, 16)
        m0 = lane == 0
        neg = jnp.full((16,), -jnp.inf, jnp.float32)

        def topk_scan(vals_ref, gidx_ref, nv, out_v_ref, out_i_ref):
            def one_pass(p, _):
                def scan4(i, carry):
                    bv, bi = carry
                    for u in range(4):
                        off = (i * 4 + u) * 16
                        x = vals_ref[pl.ds(off, 16)]
                        take = x > bv
                        bv = jnp.where(take, x, bv)
                        bi = jnp.where(take, off + lane, bi)
                    return bv, bi
                bv, bi = lax.fori_loop(0, nv // 4, scan4,
                                       (neg, jnp.zeros((16,), jnp.int32)))
                dnums = lax.GatherDimensionNumbers(
                    offset_dims=(), collapsed_slice_dims=(0,),
                    start_index_map=(0,))

                def _perm(x, pm):
                    return lax.gather(
                        x, pm[:, None], dnums, slice_sizes=(1,),
                        mode=lax.GatherScatterMode.PROMISE_IN_BOUNDS)

                for s in (8, 4, 2, 1):
                    perm = (lane + s) & 15
                    vs = _perm(bv, perm)
                    is_ = _perm(bi, perm)
                    take = vs > bv
                    bv = jnp.where(take, vs, bv)
                    bi = jnp.where(take, is_, bi)
                gv = plsc.load_gather(gidx_ref, [bi])
                pos = jnp.zeros((16,), jnp.int32) + p
                plsc.store_scatter(out_v_ref, [pos], bv, mask=m0)
                plsc.store_scatter(out_i_ref, [pos], gv, mask=m0)
                plsc.store_scatter(vals_ref, [bi], neg, mask=m0)
                return 0
            lax.fori_loop(0, nkeep, one_pass, 0)

        base = sid * pw
        pltpu.sync_copy(bmax_hbm.at[pl.ds(base, pw)], vals1)

        def fill(i, _):
            gidx1[pl.ds(i * 16, 16)] = base + i * 16 + lane
            return 0
        lax.fori_loop(0, nv1, fill, 0)
        topk_scan(vals1, gidx1, nv1, res_v, res_i)

        pltpu.sync_copy(res_v, sh_v.at[pl.ds(sid * nkeep, nkeep)])
        pltpu.sync_copy(res_i, sh_i.at[pl.ds(sid * nkeep, nkeep)])
        plsc.subcore_barrier()
        pltpu.sync_copy(sh_v, merge_v)
        pltpu.sync_copy(sh_i, merge_i)
        topk_scan(merge_v, merge_i, (ns * nkeep) // 16, blk_v, blk_i)

        pltpu.async_copy(gre_hbm.at[blk_i], grer, sem).wait()
        pltpu.async_copy(gim_hbm.at[blk_i], gimr, sem).wait()
        for t in range(rpw):
            rowv = jnp.zeros((16,), jnp.int32) + (sid * rpw + t)
            for o in range(8):
                col = o * 16 + lane
                rv = plsc.load_gather(grer, [rowv, col])
                iv = plsc.load_gather(gimr, [rowv, col])
                mvals[pl.ds((t * 8 + o) * 16, 16)] = rv * rv + iv * iv
                mgidx[pl.ds((t * 8 + o) * 16, 16)] = rowv * 128 + col
        topk_scan(mvals, mgidx, rpw * 8, res_v, res_i)

        plsc.subcore_barrier()
        pltpu.sync_copy(res_v, sh_v.at[pl.ds(sid * nkeep, nkeep)])
        pltpu.sync_copy(res_i, sh_i.at[pl.ds(sid * nkeep, nkeep)])
        plsc.subcore_barrier()
        pltpu.sync_copy(sh_v, merge_v)
        pltpu.sync_copy(sh_i, merge_i)
        topk_scan(merge_v, merge_i, (ns * nkeep) // 16, fin_v, fin_p)

        @pl.when((sid == 0) & (cid == 0))
        def _():
            for g in range(nkeep // 16):
                pv = fin_p[pl.ds(g * 16, 16)]
                rowv = pv >> 7
                offv = pv & 127
                outa[pl.ds(g * 16, 16)] = plsc.load_gather(grer, [rowv, offv])
                outb[pl.ds(g * 16, 16)] = plsc.load_gather(gimr, [rowv, offv])
                outi[pl.ds(g * 16, 16)] = (
                    plsc.load_gather(blk_i, [rowv]) * 128 + offv)
            pltpu.sync_copy(outa, a_out)
            pltpu.sync_copy(outb, b_out)
            pltpu.sync_copy(outi, i_out)

    run = functools.partial(
        pl.kernel,
        mesh=mesh,
        compiler_params=pltpu.CompilerParams(needs_layout_passes=False),
        out_type=[
            jax.ShapeDtypeStruct((nkeep,), jnp.float32),
            jax.ShapeDtypeStruct((nkeep,), jnp.float32),
            jax.ShapeDtypeStruct((nkeep,), jnp.int32),
        ],
        scratch_types=[
            pltpu.VMEM((pw,), jnp.float32),
            pltpu.VMEM((pw,), jnp.int32),
            pltpu.VMEM((nkeep,), jnp.float32),
            pltpu.VMEM((nkeep,), jnp.int32),
            pltpu.VMEM((ns * nkeep,), jnp.float32),
            pltpu.VMEM((ns * nkeep,), jnp.int32),
            pltpu.VMEM((nkeep,), jnp.float32),
            pltpu.VMEM((nkeep,), jnp.int32),
            pltpu.VMEM((nkeep, 128), jnp.float32),
            pltpu.VMEM((nkeep, 128), jnp.float32),
            pltpu.VMEM((rpw * 128,), jnp.float32),
            pltpu.VMEM((rpw * 128,), jnp.int32),
            pltpu.VMEM((nkeep,), jnp.float32),
            pltpu.VMEM((nkeep,), jnp.int32),
            pltpu.VMEM((nkeep,), jnp.float32),
            pltpu.VMEM((nkeep,), jnp.float32),
            pltpu.VMEM((nkeep,), jnp.int32),
            pltpu.VMEM_SHARED((ns * nkeep,), jnp.float32),
            pltpu.VMEM_SHARED((ns * nkeep,), jnp.int32),
            pltpu.SemaphoreType.DMA,
        ],
    )(body)
    return run(bmax_flat, gre2, gim2)


N_KEEP = 64


def kernel(weight_matrix):
    n0, n1 = weight_matrix.shape
    ncp = _round_up(n1 // 2 + 1, LANE)
    bc, bs, c0, s0 = _dft_tables(n0, n1, ncp)
    bc = jnp.asarray(bc)
    bs = jnp.asarray(bs)
    c0 = jnp.asarray(c0)
    s0 = jnp.asarray(s0)

    fre, fim = _rowfft(weight_matrix, bc, bs)
    gre, gim, bmax = _colfft(c0, s0, fre, fim)

    nblocks = n0 * (ncp // LANE)
    a, b, idx = _sc_topk(bmax.reshape(-1),
                         gre.reshape(nblocks, LANE),
                         gim.reshape(nblocks, LANE),
                         nblocks, N_KEEP)
    r = idx // ncp
    c = idx % ncp

    u, v = _basis(c0, s0, a.reshape(1, -1), b.reshape(1, -1),
                  r.reshape(1, -1), c.reshape(1, -1))
    return _recon(u, v)


# speedup vs baseline: 9.9659x; 1.9559x over previous
import functools

import jax
import jax.numpy as jnp
import numpy as np
from jax import lax
from jax.experimental import pallas as pl
from jax.experimental.pallas import tpu as pltpu
from jax.experimental.pallas import tpu_sc as plsc

LANE = 128


def _round_up(x, m):
    return (x + m - 1) // m * m


@functools.lru_cache(maxsize=2)
def _dft_tables(n0: int, n1: int, ncp: int):
    nc = n1 // 2 + 1
    k = np.arange(ncp)
    n = np.arange(n1)
    ang1 = 2.0 * np.pi * ((np.outer(n, k) % n1) / n1)
    bc = np.cos(ang1)
    bs = np.sin(ang1)
    bc[:, nc:] = 0.0
    bs[:, nc:] = 0.0
    j = np.arange(n0)
    ang0 = 2.0 * np.pi * ((np.outer(j, j) % n0) / n0)
    c0 = np.cos(ang0).astype(np.float32)
    s0 = np.sin(ang0).astype(np.float32)
    return (bc.astype(np.float32), bs.astype(np.float32), c0, s0)


def _rowfft_body(w_ref, bc_ref, bs_ref, fre_ref, fim_ref):
    @pl.when(pl.program_id(1) == 0)
    def _():
        fre_ref[...] = jnp.zeros_like(fre_ref)
        fim_ref[...] = jnp.zeros_like(fim_ref)

    w = w_ref[...]
    fre_ref[...] += jax.lax.dot(w, bc_ref[...],
                                precision=jax.lax.Precision.HIGHEST,
                                preferred_element_type=jnp.float32)
    fim_ref[...] += jax.lax.dot(w, -bs_ref[...],
                                precision=jax.lax.Precision.HIGHEST,
                                preferred_element_type=jnp.float32)


def _rowfft(w, bc, bs, bm=256, bk=256):
    n0, n1 = w.shape
    bm, bk = min(bm, n0), min(bk, n1)
    ncp = bc.shape[1]
    grid = (n0 // bm, n1 // bk)
    return pl.pallas_call(
        _rowfft_body,
        grid=grid,
        in_specs=[
            pl.BlockSpec((bm, bk), lambda i, k: (i, k)),
            pl.BlockSpec((bk, ncp), lambda i, k: (k, 0)),
            pl.BlockSpec((bk, ncp), lambda i, k: (k, 0)),
        ],
        out_specs=[
            pl.BlockSpec((bm, ncp), lambda i, k: (i, 0)),
            pl.BlockSpec((bm, ncp), lambda i, k: (i, 0)),
        ],
        out_shape=[
            jax.ShapeDtypeStruct((n0, ncp), jnp.float32),
            jax.ShapeDtypeStruct((n0, ncp), jnp.float32),
        ],
    )(w, bc, bs)


@functools.lru_cache(maxsize=2)
def _ct_tables(n0: int):
    f = int(round(np.sqrt(n0)))
    assert f * f == n0
    bqa = min(f, max(8, 256 // f))
    bqb = max(1, 256 // f)
    j1 = np.arange(f)
    r1 = np.arange(f)
    la = np.zeros((f // bqa, f * bqa, f * bqa), dtype=np.complex128)
    for g in range(f // bqa):
        for q in range(bqa):
            ang = np.outer(j1, f * r1 + g * bqa + q) * (2.0 * np.pi / n0)
            la[g, q::bqa, q::bqa] = np.exp(-1j * ang)
    j2 = np.arange(f)
    wf = np.exp(-2j * np.pi * np.outer(j2, j2) / f)
    lb = np.kron(np.eye(bqb), wf)
    return (la.real.astype(np.float32), la.imag.astype(np.float32),
            lb.real.astype(np.float32), lb.imag.astype(np.float32),
            f, bqa, bqb)


def _ct_stage_a_body(lare_ref, laim_ref, fre_ref, fim_ref, tre_ref, tim_ref):
    rows = lare_ref.shape[1]
    ncp = fre_ref.shape[2]
    la_re = lare_ref[...].reshape(rows, rows)
    la_im = laim_ref[...].reshape(rows, rows)
    f_re = fre_ref[...].reshape(rows, ncp)
    f_im = fim_ref[...].reshape(rows, ncp)
    hi = jax.lax.Precision.HIGHEST
    t_re = (jax.lax.dot(la_re, f_re, precision=hi,
                        preferred_element_type=jnp.float32)
            - jax.lax.dot(la_im, f_im, precision=hi,
                          preferred_element_type=jnp.float32))
    t_im = (jax.lax.dot(la_re, f_im, precision=hi,
                        preferred_element_type=jnp.float32)
            + jax.lax.dot(la_im, f_re, precision=hi,
                          preferred_element_type=jnp.float32))
    tre_ref[...] = t_re.reshape(tre_ref.shape)
    tim_ref[...] = t_im.reshape(tim_ref.shape)


def _ct_stage_b_body(lbre_ref, lbim_ref, tre_ref, tim_ref,
                     zre_ref, zim_ref, bmax_ref, *, nblk):
    rows = lbre_ref.shape[0]
    ncp = tre_ref.shape[2]
    lb_re = lbre_ref[...]
    lb_im = lbim_ref[...]
    t_re = tre_ref[...].reshape(rows, ncp)
    t_im = tim_ref[...].reshape(rows, ncp)
    hi = jax.lax.Precision.HIGHEST
    z_re = (jax.lax.dot(lb_re, t_re, precision=hi,
                        preferred_element_type=jnp.float32)
            - jax.lax.dot(lb_im, t_im, precision=hi,
                          preferred_element_type=jnp.float32))
    z_im = (jax.lax.dot(lb_re, t_im, precision=hi,
                        preferred_element_type=jnp.float32)
            + jax.lax.dot(lb_im, t_re, precision=hi,
                          preferred_element_type=jnp.float32))
    mag2 = z_re * z_re + z_im * z_im
    for j in range(nblk):
        bmax_ref[:, j] = jnp.max(mag2[:, j * LANE:(j + 1) * LANE], axis=1)
    zre_ref[...] = z_re.reshape(zre_ref.shape)
    zim_ref[...] = z_im.reshape(zim_ref.shape)


def _colfft_ct(fre, fim):
    n0, ncp = fre.shape
    lar, lai, lbr, lbi, f, bqa, bqb = _ct_tables(n0)
    lar, lai = jnp.asarray(lar), jnp.asarray(lai)
    lbr, lbi = jnp.asarray(lbr), jnp.asarray(lbi)
    nblk = ncp // LANE
    f3 = (f, f, ncp)
    fre3 = fre.reshape(f3)
    fim3 = fim.reshape(f3)
    nct = 1 if ncp <= 1280 else 2
    cta = ncp // nct
    tre, tim = pl.pallas_call(
        _ct_stage_a_body,
        grid=(f // bqa, nct),
        in_specs=[
            pl.BlockSpec((1, f * bqa, f * bqa), lambda g, t: (g, 0, 0)),
            pl.BlockSpec((1, f * bqa, f * bqa), lambda g, t: (g, 0, 0)),
            pl.BlockSpec((f, bqa, cta), lambda g, t: (0, g, t)),
            pl.BlockSpec((f, bqa, cta), lambda g, t: (0, g, t)),
        ],
        out_specs=[
            pl.BlockSpec((f, bqa, cta), lambda g, t: (0, g, t)),
            pl.BlockSpec((f, bqa, cta), lambda g, t: (0, g, t)),
        ],
        out_shape=[
            jax.ShapeDtypeStruct(f3, jnp.float32),
            jax.ShapeDtypeStruct(f3, jnp.float32),
        ],
    )(lar, lai, fre3, fim3)
    zre, zim, bmax = pl.pallas_call(
        functools.partial(_ct_stage_b_body, nblk=nblk),
        grid=(f // bqb,),
        in_specs=[
            pl.BlockSpec((f * bqb, f * bqb), lambda g: (0, 0)),
            pl.BlockSpec((f * bqb, f * bqb), lambda g: (0, 0)),
            pl.BlockSpec((bqb, f, ncp), lambda g: (g, 0, 0)),
            pl.BlockSpec((bqb, f, ncp), lambda g: (g, 0, 0)),
        ],
        out_specs=[
            pl.BlockSpec((bqb, f, ncp), lambda g: (g, 0, 0)),
            pl.BlockSpec((bqb, f, ncp), lambda g: (g, 0, 0)),
            pl.BlockSpec((f * bqb, nblk), lambda g: (g, 0)),
        ],
        out_shape=[
            jax.ShapeDtypeStruct(f3, jnp.float32),
            jax.ShapeDtypeStruct(f3, jnp.float32),
            jax.ShapeDtypeStruct((n0, nblk), jnp.float32),
        ],
    )(lbr, lbi, tre, tim)
    return (zre.reshape(n0, ncp), zim.reshape(n0, ncp), bmax, f)


def _colfft_body(c0_ref, s0_ref, fre_ref, fim_ref,
                 gre_ref, gim_ref, bmax_ref, *, nblk):
    @pl.when(pl.program_id(1) == 0)
    def _():
        gre_ref[...] = jnp.zeros_like(gre_ref)
        gim_ref[...] = jnp.zeros_like(gim_ref)

    c0 = c0_ref[...]
    s0 = s0_ref[...]
    fre = fre_ref[...]
    fim = fim_ref[...]
    hi = jax.lax.Precision.HIGHEST
    gre_ref[...] += (jax.lax.dot(c0, fre, precision=hi,
                                 preferred_element_type=jnp.float32)
                     + jax.lax.dot(s0, fim, precision=hi,
                                   preferred_element_type=jnp.float32))
    gim_ref[...] += (jax.lax.dot(c0, fim, precision=hi,
                                 preferred_element_type=jnp.float32)
                     - jax.lax.dot(s0, fre, precision=hi,
                                   preferred_element_type=jnp.float32))

    @pl.when(pl.program_id(1) == pl.num_programs(1) - 1)
    def _():
        gre = gre_ref[...]
        gim = gim_ref[...]
        mag2 = gre * gre + gim * gim
        for j in range(nblk):
            blk = mag2[:, j * LANE:(j + 1) * LANE]
            bmax_ref[:, j] = jnp.max(blk, axis=1)


def _colfft(c0, s0, fre, fim, bm=256, bk=256):
    n0 = c0.shape[0]
    bm, bk = min(bm, n0), min(bk, n0)
    ncp = fre.shape[1]
    nblk = ncp // LANE
    grid = (n0 // bm, n0 // bk)
    return pl.pallas_call(
        functools.partial(_colfft_body, nblk=nblk),
        grid=grid,
        in_specs=[
            pl.BlockSpec((bm, bk), lambda i, k: (i, k)),
            pl.BlockSpec((bm, bk), lambda i, k: (i, k)),
            pl.BlockSpec((bk, ncp), lambda i, k: (k, 0)),
            pl.BlockSpec((bk, ncp), lambda i, k: (k, 0)),
        ],
        out_specs=[
            pl.BlockSpec((bm, ncp), lambda i, k: (i, 0)),
            pl.BlockSpec((bm, ncp), lambda i, k: (i, 0)),
            pl.BlockSpec((bm, nblk), lambda i, k: (i, 0)),
        ],
        out_shape=[
            jax.ShapeDtypeStruct((n0, ncp), jnp.float32),
            jax.ShapeDtypeStruct((n0, ncp), jnp.float32),
            jax.ShapeDtypeStruct((n0, nblk), jnp.float32),
        ],
    )(c0, s0, fre, fim)


def _basis_body(c0_ref, s0_ref, a_ref, b_ref, r_ref, c_ref,
                u_ref, v_ref, *, n0, n1, nmodes):
    rows = jax.lax.broadcasted_iota(jnp.int32, (c0_ref.shape[0], 1), 0) \
        + pl.program_id(0) * c0_ref.shape[0]
    r = r_ref[0, :]
    c = c_ref[0, :]
    a = a_ref[0, :]
    b = b_ref[0, :]
    onehot_r = (jax.lax.broadcasted_iota(jnp.int32, (c0_ref.shape[1], nmodes), 0)
                == r[None, :]).astype(jnp.float32)
    onehot_c = (jax.lax.broadcasted_iota(jnp.int32, (c0_ref.shape[1], nmodes), 0)
                == c[None, :]).astype(jnp.bfloat16)
    onehot_r = onehot_r.astype(jnp.bfloat16)

    def _gdot(x, oh):
        xh = x.astype(jnp.bfloat16)
        xl = (x - xh.astype(jnp.float32)).astype(jnp.bfloat16)
        return (jax.lax.dot(xh, oh, preferred_element_type=jnp.float32)
                + jax.lax.dot(xl, oh, preferred_element_type=jnp.float32))

    cr = _gdot(c0_ref[...], onehot_r)
    sr = _gdot(s0_ref[...], onehot_r)
    cc = _gdot(c0_ref[...], onehot_c)
    sc = _gdot(s0_ref[...], onehot_c)
    del rows
    p = a[None, :] * cr - b[None, :] * sr
    q = -(a[None, :] * sr + b[None, :] * cr)
    u_ref[...] = jnp.concatenate([p, q], axis=1)
    w = jnp.where((c == 0) | (c == n1 // 2), 1.0, 2.0) / (
        jnp.float32(n0) * jnp.float32(n1))
    v_ref[...] = jnp.concatenate([w[None, :] * cc, w[None, :] * sc], axis=1)


def _basis(c0, s0, a, b, r, c, bm=256):
    n0 = c0.shape[0]
    bm = min(bm, n0)
    nmodes = a.shape[1]
    grid = (n0 // bm,)
    return pl.pallas_call(
        functools.partial(_basis_body, n0=n0, n1=n0, nmodes=nmodes),
        grid=grid,
        in_specs=[
            pl.BlockSpec((bm, n0), lambda i: (i, 0)),
            pl.BlockSpec((bm, n0), lambda i: (i, 0)),
            pl.BlockSpec((1, nmodes), lambda i: (0, 0)),
            pl.BlockSpec((1, nmodes), lambda i: (0, 0)),
            pl.BlockSpec((1, nmodes), lambda i: (0, 0)),
            pl.BlockSpec((1, nmodes), lambda i: (0, 0)),
        ],
        out_specs=[
            pl.BlockSpec((bm, 2 * nmodes), lambda i: (i, 0)),
            pl.BlockSpec((bm, 2 * nmodes), lambda i: (i, 0)),
        ],
        out_shape=[
            jax.ShapeDtypeStruct((n0, 2 * nmodes), jnp.float32),
            jax.ShapeDtypeStruct((n0, 2 * nmodes), jnp.float32),
        ],
    )(c0, s0, a, b, r, c)


def _recon_body(u_ref, v_ref, out_ref):
    out_ref[...] = jax.lax.dot_general(
        u_ref[...], v_ref[...],
        dimension_numbers=(((1,), (1,)), ((), ())),
        precision=jax.lax.Precision.HIGHEST,
        preferred_element_type=jnp.float32)


def _recon(u, v, bm=512, bn=512):
    n0 = u.shape[0]
    bm, bn = min(bm, n0), min(bn, n0)
    kk = u.shape[1]
    grid = (n0 // bm, n0 // bn)
    return pl.pallas_call(
        _recon_body,
        grid=grid,
        in_specs=[
            pl.BlockSpec((bm, kk), lambda i, j: (i, 0)),
            pl.BlockSpec((bn, kk), lambda i, j: (j, 0)),
        ],
        out_specs=pl.BlockSpec((bm, bn), lambda i, j: (i, j)),
        out_shape=jax.ShapeDtypeStruct((n0, n0), jnp.float32),
    )(u, v)


def _sc_topk(bmax_flat, gre2, gim2, nblocks, nkeep):
    ns = 16
    pw = nblocks // ns
    nv1 = pw // 16
    rpw = nkeep // ns
    mesh = plsc.VectorSubcoreMesh(core_axis_name="c", subcore_axis_name="s")

    def body(bmax_hbm, gre_hbm, gim_hbm, a_out, b_out, i_out,
             vals1, gidx1, res_v, res_i, merge_v, merge_i, blk_v, blk_i,
             grer, gimr, mvals, mgidx, fin_v, fin_p, outa, outb, outi,
             sh_v, sh_i, sem):
        sid = lax.axis_index("s")
        cid = lax.axis_index("c")
        lane = lax.iota(jnp.int32, 16)
        m0 = lane == 0
        neg = jnp.full((16,), -jnp.inf, jnp.float32)

        def topk_scan(vals_ref, gidx_ref, nv, out_v_ref, out_i_ref):
            def one_pass(p, _):
                def scan4(i, carry):
                    bv, bi = carry
                    for u in range(4):
                        off = (i * 4 + u) * 16
                        x = vals_ref[pl.ds(off, 16)]
                        take = x > bv
                        bv = jnp.where(take, x, bv)
                        bi = jnp.where(take, off + lane, bi)
                    return bv, bi
                bv, bi = lax.fori_loop(0, nv // 4, scan4,
                                       (neg, jnp.zeros((16,), jnp.int32)))
                dnums = lax.GatherDimensionNumbers(
                    offset_dims=(), collapsed_slice_dims=(0,),
                    start_index_map=(0,))

                def _perm(x, pm):
                    return lax.gather(
                        x, pm[:, None], dnums, slice_sizes=(1,),
                        mode=lax.GatherScatterMode.PROMISE_IN_BOUNDS)

                for s in (8, 4, 2, 1):
                    perm = (lane + s) & 15
                    vs = _perm(bv, perm)
                    is_ = _perm(bi, perm)
                    take = vs > bv
                    bv = jnp.where(take, vs, bv)
                    bi = jnp.where(take, is_, bi)
                gv = plsc.load_gather(gidx_ref, [bi])
                pos = jnp.zeros((16,), jnp.int32) + p
                plsc.store_scatter(out_v_ref, [pos], bv, mask=m0)
                plsc.store_scatter(out_i_ref, [pos], gv, mask=m0)
                plsc.store_scatter(vals_ref, [bi], neg, mask=m0)
                return 0
            lax.fori_loop(0, nkeep, one_pass, 0)

        base = sid * pw
        pltpu.sync_copy(bmax_hbm.at[pl.ds(base, pw)], vals1)

        def fill(i, _):
            gidx1[pl.ds(i * 16, 16)] = base + i * 16 + lane
            return 0
        lax.fori_loop(0, nv1, fill, 0)
        topk_scan(vals1, gidx1, nv1, res_v, res_i)

        pltpu.sync_copy(res_v, sh_v.at[pl.ds(sid * nkeep, nkeep)])
        pltpu.sync_copy(res_i, sh_i.at[pl.ds(sid * nkeep, nkeep)])
        plsc.subcore_barrier()
        pltpu.sync_copy(sh_v, merge_v)
        pltpu.sync_copy(sh_i, merge_i)
        topk_scan(merge_v, merge_i, (ns * nkeep) // 16, blk_v, blk_i)

        pltpu.async_copy(gre_hbm.at[blk_i], grer, sem).wait()
        pltpu.async_copy(gim_hbm.at[blk_i], gimr, sem).wait()
        for t in range(rpw):
            rowv = jnp.zeros((16,), jnp.int32) + (sid * rpw + t)
            for o in range(8):
                col = o * 16 + lane
                rv = plsc.load_gather(grer, [rowv, col])
                iv = plsc.load_gather(gimr, [rowv, col])
                mvals[pl.ds((t * 8 + o) * 16, 16)] = rv * rv + iv * iv
                mgidx[pl.ds((t * 8 + o) * 16, 16)] = rowv * 128 + col
        topk_scan(mvals, mgidx, rpw * 8, res_v, res_i)

        plsc.subcore_barrier()
        pltpu.sync_copy(res_v, sh_v.at[pl.ds(sid * nkeep, nkeep)])
        pltpu.sync_copy(res_i, sh_i.at[pl.ds(sid * nkeep, nkeep)])
        plsc.subcore_barrier()
        pltpu.sync_copy(sh_v, merge_v)
        pltpu.sync_copy(sh_i, merge_i)
        topk_scan(merge_v, merge_i, (ns * nkeep) // 16, fin_v, fin_p)

        @pl.when((sid == 0) & (cid == 0))
        def _():
            for g in range(nkeep // 16):
                pv = fin_p[pl.ds(g * 16, 16)]
                rowv = pv >> 7
                offv = pv & 127
                outa[pl.ds(g * 16, 16)] = plsc.load_gather(grer, [rowv, offv])
                outb[pl.ds(g * 16, 16)] = plsc.load_gather(gimr, [rowv, offv])
                outi[pl.ds(g * 16, 16)] = (
                    plsc.load_gather(blk_i, [rowv]) * 128 + offv)
            pltpu.sync_copy(outa, a_out)
            pltpu.sync_copy(outb, b_out)
            pltpu.sync_copy(outi, i_out)

    run = functools.partial(
        pl.kernel,
        mesh=mesh,
        compiler_params=pltpu.CompilerParams(needs_layout_passes=False),
        out_type=[
            jax.ShapeDtypeStruct((nkeep,), jnp.float32),
            jax.ShapeDtypeStruct((nkeep,), jnp.float32),
            jax.ShapeDtypeStruct((nkeep,), jnp.int32),
        ],
        scratch_types=[
            pltpu.VMEM((pw,), jnp.float32),
            pltpu.VMEM((pw,), jnp.int32),
            pltpu.VMEM((nkeep,), jnp.float32),
            pltpu.VMEM((nkeep,), jnp.int32),
            pltpu.VMEM((ns * nkeep,), jnp.float32),
            pltpu.VMEM((ns * nkeep,), jnp.int32),
            pltpu.VMEM((nkeep,), jnp.float32),
            pltpu.VMEM((nkeep,), jnp.int32),
            pltpu.VMEM((nkeep, 128), jnp.float32),
            pltpu.VMEM((nkeep, 128), jnp.float32),
            pltpu.VMEM((rpw * 128,), jnp.float32),
            pltpu.VMEM((rpw * 128,), jnp.int32),
            pltpu.VMEM((nkeep,), jnp.float32),
            pltpu.VMEM((nkeep,), jnp.int32),
            pltpu.VMEM((nkeep,), jnp.float32),
            pltpu.VMEM((nkeep,), jnp.float32),
            pltpu.VMEM((nkeep,), jnp.int32),
            pltpu.VMEM_SHARED((ns * nkeep,), jnp.float32),
            pltpu.VMEM_SHARED((ns * nkeep,), jnp.int32),
            pltpu.SemaphoreType.DMA,
        ],
    )(body)
    return run(bmax_flat, gre2, gim2)


N_KEEP = 64


def kernel(weight_matrix):
    n0, n1 = weight_matrix.shape
    ncp = _round_up(n1 // 2 + 1, LANE)
    if ncp > 1280:
        ncp = _round_up(n1 // 2 + 1, 2 * LANE)
    bc, bs, c0, s0 = _dft_tables(n0, n1, ncp)
    bc = jnp.asarray(bc)
    bs = jnp.asarray(bs)
    c0 = jnp.asarray(c0)
    s0 = jnp.asarray(s0)

    fre, fim = _rowfft(weight_matrix, bc, bs)
    gre, gim, bmax, fct = _colfft_ct(fre, fim)

    nblocks = n0 * (ncp // LANE)
    a, b, idx = _sc_topk(bmax.reshape(-1),
                         gre.reshape(nblocks, LANE),
                         gim.reshape(nblocks, LANE),
                         nblocks, N_KEEP)
    rp = idx // ncp
    r = (rp % fct) * fct + rp // fct
    c = idx % ncp

    u, v = _basis(c0, s0, a.reshape(1, -1), b.reshape(1, -1),
                  r.reshape(1, -1), c.reshape(1, -1))
    return _recon(u, v)


# speedup vs baseline: 10.3174x; 1.0353x over previous
import functools

import jax
import jax.numpy as jnp
import numpy as np
from jax import lax
from jax.experimental import pallas as pl
from jax.experimental.pallas import tpu as pltpu
from jax.experimental.pallas import tpu_sc as plsc

LANE = 128


def _round_up(x, m):
    return (x + m - 1) // m * m


@functools.lru_cache(maxsize=2)
def _dft_tables(n0: int, n1: int, ncp: int):
    nc = n1 // 2 + 1
    k = np.arange(ncp)
    n = np.arange(n1)
    ang1 = 2.0 * np.pi * ((np.outer(n, k) % n1) / n1)
    bc = np.cos(ang1)
    bs = np.sin(ang1)
    bc[:, nc:] = 0.0
    bs[:, nc:] = 0.0
    j = np.arange(n0)
    ang0 = 2.0 * np.pi * ((np.outer(j, j) % n0) / n0)
    c0 = np.cos(ang0).astype(np.float32)
    s0 = np.sin(ang0).astype(np.float32)
    return (bc.astype(np.float32), bs.astype(np.float32), c0, s0)


def _rowfft_body(w_ref, bc_ref, bs_ref, fre_ref, fim_ref):
    @pl.when(pl.program_id(1) == 0)
    def _():
        fre_ref[...] = jnp.zeros_like(fre_ref)
        fim_ref[...] = jnp.zeros_like(fim_ref)

    w = w_ref[...]
    fre_ref[...] += jax.lax.dot(w, bc_ref[...],
                                precision=jax.lax.Precision.HIGHEST,
                                preferred_element_type=jnp.float32)
    fim_ref[...] += jax.lax.dot(w, -bs_ref[...],
                                precision=jax.lax.Precision.HIGHEST,
                                preferred_element_type=jnp.float32)


def _rowfft(w, bc, bs, bm=512, bk=256):
    n0, n1 = w.shape
    bm, bk = min(bm, n0), min(bk, n1)
    ncp = bc.shape[1]
    grid = (n0 // bm, n1 // bk)
    return pl.pallas_call(
        _rowfft_body,
        grid=grid,
        in_specs=[
            pl.BlockSpec((bm, bk), lambda i, k: (i, k)),
            pl.BlockSpec((bk, ncp), lambda i, k: (k, 0)),
            pl.BlockSpec((bk, ncp), lambda i, k: (k, 0)),
        ],
        out_specs=[
            pl.BlockSpec((bm, ncp), lambda i, k: (i, 0)),
            pl.BlockSpec((bm, ncp), lambda i, k: (i, 0)),
        ],
        out_shape=[
            jax.ShapeDtypeStruct((n0, ncp), jnp.float32),
            jax.ShapeDtypeStruct((n0, ncp), jnp.float32),
        ],
    )(w, bc, bs)


@functools.lru_cache(maxsize=2)
def _ct_tables(n0: int):
    f = int(round(np.sqrt(n0)))
    assert f * f == n0
    bqa = min(f, max(8, 256 // f))
    bqb = max(1, 256 // f)
    j1 = np.arange(f)
    r1 = np.arange(f)
    la = np.zeros((f // bqa, f * bqa, f * bqa), dtype=np.complex128)
    for g in range(f // bqa):
        for q in range(bqa):
            ang = np.outer(j1, f * r1 + g * bqa + q) * (2.0 * np.pi / n0)
            la[g, q::bqa, q::bqa] = np.exp(-1j * ang)
    j2 = np.arange(f)
    wf = np.exp(-2j * np.pi * np.outer(j2, j2) / f)
    lb = np.kron(np.eye(bqb), wf)
    return (la.real.astype(np.float32), la.imag.astype(np.float32),
            lb.real.astype(np.float32), lb.imag.astype(np.float32),
            f, bqa, bqb)


def _ct_stage_a_body(lare_ref, laim_ref, fre_ref, fim_ref, tre_ref, tim_ref):
    rows = lare_ref.shape[1]
    ncp = fre_ref.shape[2]
    la_re = lare_ref[...].reshape(rows, rows)
    la_im = laim_ref[...].reshape(rows, rows)
    f_re = fre_ref[...].reshape(rows, ncp)
    f_im = fim_ref[...].reshape(rows, ncp)
    hi = jax.lax.Precision.HIGHEST
    t_re = (jax.lax.dot(la_re, f_re, precision=hi,
                        preferred_element_type=jnp.float32)
            - jax.lax.dot(la_im, f_im, precision=hi,
                          preferred_element_type=jnp.float32))
    t_im = (jax.lax.dot(la_re, f_im, precision=hi,
                        preferred_element_type=jnp.float32)
            + jax.lax.dot(la_im, f_re, precision=hi,
                          preferred_element_type=jnp.float32))
    tre_ref[...] = t_re.reshape(tre_ref.shape)
    tim_ref[...] = t_im.reshape(tim_ref.shape)


def _ct_stage_b_body(lbre_ref, lbim_ref, tre_ref, tim_ref,
                     zre_ref, zim_ref, bmax_ref, *, nblk):
    rows = lbre_ref.shape[0]
    ncp = tre_ref.shape[2]
    lb_re = lbre_ref[...]
    lb_im = lbim_ref[...]
    t_re = tre_ref[...].reshape(rows, ncp)
    t_im = tim_ref[...].reshape(rows, ncp)
    hi = jax.lax.Precision.HIGHEST
    z_re = (jax.lax.dot(lb_re, t_re, precision=hi,
                        preferred_element_type=jnp.float32)
            - jax.lax.dot(lb_im, t_im, precision=hi,
                          preferred_element_type=jnp.float32))
    z_im = (jax.lax.dot(lb_re, t_im, precision=hi,
                        preferred_element_type=jnp.float32)
            + jax.lax.dot(lb_im, t_re, precision=hi,
                          preferred_element_type=jnp.float32))
    mag2 = z_re * z_re + z_im * z_im
    for j in range(nblk):
        bmax_ref[:, j] = jnp.max(mag2[:, j * LANE:(j + 1) * LANE], axis=1)
    zre_ref[...] = z_re.reshape(zre_ref.shape)
    zim_ref[...] = z_im.reshape(zim_ref.shape)


def _colfft_ct(fre, fim):
    n0, ncp = fre.shape
    lar, lai, lbr, lbi, f, bqa, bqb = _ct_tables(n0)
    lar, lai = jnp.asarray(lar), jnp.asarray(lai)
    lbr, lbi = jnp.asarray(lbr), jnp.asarray(lbi)
    nblk = ncp // LANE
    f3 = (f, f, ncp)
    fre3 = fre.reshape(f3)
    fim3 = fim.reshape(f3)
    nct = 1 if ncp <= 1280 else 2
    cta = ncp // nct
    tre, tim = pl.pallas_call(
        _ct_stage_a_body,
        grid=(f // bqa, nct),
        in_specs=[
            pl.BlockSpec((1, f * bqa, f * bqa), lambda g, t: (g, 0, 0)),
            pl.BlockSpec((1, f * bqa, f * bqa), lambda g, t: (g, 0, 0)),
            pl.BlockSpec((f, bqa, cta), lambda g, t: (0, g, t)),
            pl.BlockSpec((f, bqa, cta), lambda g, t: (0, g, t)),
        ],
        out_specs=[
            pl.BlockSpec((f, bqa, cta), lambda g, t: (0, g, t)),
            pl.BlockSpec((f, bqa, cta), lambda g, t: (0, g, t)),
        ],
        out_shape=[
            jax.ShapeDtypeStruct(f3, jnp.float32),
            jax.ShapeDtypeStruct(f3, jnp.float32),
        ],
    )(lar, lai, fre3, fim3)
    zre, zim, bmax = pl.pallas_call(
        functools.partial(_ct_stage_b_body, nblk=nblk),
        grid=(f // bqb,),
        in_specs=[
            pl.BlockSpec((f * bqb, f * bqb), lambda g: (0, 0)),
            pl.BlockSpec((f * bqb, f * bqb), lambda g: (0, 0)),
            pl.BlockSpec((bqb, f, ncp), lambda g: (g, 0, 0)),
            pl.BlockSpec((bqb, f, ncp), lambda g: (g, 0, 0)),
        ],
        out_specs=[
            pl.BlockSpec((bqb, f, ncp), lambda g: (g, 0, 0)),
            pl.BlockSpec((bqb, f, ncp), lambda g: (g, 0, 0)),
            pl.BlockSpec((f * bqb, nblk), lambda g: (g, 0)),
        ],
        out_shape=[
            jax.ShapeDtypeStruct(f3, jnp.float32),
            jax.ShapeDtypeStruct(f3, jnp.float32),
            jax.ShapeDtypeStruct((n0, nblk), jnp.float32),
        ],
    )(lbr, lbi, tre, tim)
    return (zre.reshape(n0, ncp), zim.reshape(n0, ncp), bmax, f)


def _colfft_body(c0_ref, s0_ref, fre_ref, fim_ref,
                 gre_ref, gim_ref, bmax_ref, *, nblk):
    @pl.when(pl.program_id(1) == 0)
    def _():
        gre_ref[...] = jnp.zeros_like(gre_ref)
        gim_ref[...] = jnp.zeros_like(gim_ref)

    c0 = c0_ref[...]
    s0 = s0_ref[...]
    fre = fre_ref[...]
    fim = fim_ref[...]
    hi = jax.lax.Precision.HIGHEST
    gre_ref[...] += (jax.lax.dot(c0, fre, precision=hi,
                                 preferred_element_type=jnp.float32)
                     + jax.lax.dot(s0, fim, precision=hi,
                                   preferred_element_type=jnp.float32))
    gim_ref[...] += (jax.lax.dot(c0, fim, precision=hi,
                                 preferred_element_type=jnp.float32)
                     - jax.lax.dot(s0, fre, precision=hi,
                                   preferred_element_type=jnp.float32))

    @pl.when(pl.program_id(1) == pl.num_programs(1) - 1)
    def _():
        gre = gre_ref[...]
        gim = gim_ref[...]
        mag2 = gre * gre + gim * gim
        for j in range(nblk):
            blk = mag2[:, j * LANE:(j + 1) * LANE]
            bmax_ref[:, j] = jnp.max(blk, axis=1)


def _colfft(c0, s0, fre, fim, bm=256, bk=256):
    n0 = c0.shape[0]
    bm, bk = min(bm, n0), min(bk, n0)
    ncp = fre.shape[1]
    nblk = ncp // LANE
    grid = (n0 // bm, n0 // bk)
    return pl.pallas_call(
        functools.partial(_colfft_body, nblk=nblk),
        grid=grid,
        in_specs=[
            pl.BlockSpec((bm, bk), lambda i, k: (i, k)),
            pl.BlockSpec((bm, bk), lambda i, k: (i, k)),
            pl.BlockSpec((bk, ncp), lambda i, k: (k, 0)),
            pl.BlockSpec((bk, ncp), lambda i, k: (k, 0)),
        ],
        out_specs=[
            pl.BlockSpec((bm, ncp), lambda i, k: (i, 0)),
            pl.BlockSpec((bm, ncp), lambda i, k: (i, 0)),
            pl.BlockSpec((bm, nblk), lambda i, k: (i, 0)),
        ],
        out_shape=[
            jax.ShapeDtypeStruct((n0, ncp), jnp.float32),
            jax.ShapeDtypeStruct((n0, ncp), jnp.float32),
            jax.ShapeDtypeStruct((n0, nblk), jnp.float32),
        ],
    )(c0, s0, fre, fim)


def _basis_body(c0_ref, s0_ref, a_ref, b_ref, r_ref, c_ref,
                u_ref, v_ref, *, n0, n1, nmodes):
    rows = jax.lax.broadcasted_iota(jnp.int32, (c0_ref.shape[0], 1), 0) \
        + pl.program_id(0) * c0_ref.shape[0]
    r = r_ref[0, :]
    c = c_ref[0, :]
    a = a_ref[0, :]
    b = b_ref[0, :]
    onehot_r = (jax.lax.broadcasted_iota(jnp.int32, (c0_ref.shape[1], nmodes), 0)
                == r[None, :]).astype(jnp.float32)
    onehot_c = (jax.lax.broadcasted_iota(jnp.int32, (c0_ref.shape[1], nmodes), 0)
                == c[None, :]).astype(jnp.bfloat16)
    onehot_r = onehot_r.astype(jnp.bfloat16)

    def _gdot(x, oh):
        xh = x.astype(jnp.bfloat16)
        xl = (x - xh.astype(jnp.float32)).astype(jnp.bfloat16)
        return (jax.lax.dot(xh, oh, preferred_element_type=jnp.float32)
                + jax.lax.dot(xl, oh, preferred_element_type=jnp.float32))

    cr = _gdot(c0_ref[...], onehot_r)
    sr = _gdot(s0_ref[...], onehot_r)
    cc = _gdot(c0_ref[...], onehot_c)
    sc = _gdot(s0_ref[...], onehot_c)
    del rows
    p = a[None, :] * cr - b[None, :] * sr
    q = -(a[None, :] * sr + b[None, :] * cr)
    u_ref[...] = jnp.concatenate([p, q], axis=1)
    w = jnp.where((c == 0) | (c == n1 // 2), 1.0, 2.0) / (
        jnp.float32(n0) * jnp.float32(n1))
    v_ref[...] = jnp.concatenate([w[None, :] * cc, w[None, :] * sc], axis=1)


def _basis(c0, s0, a, b, r, c, bm=256):
    n0 = c0.shape[0]
    bm = min(bm, n0)
    nmodes = a.shape[1]
    grid = (n0 // bm,)
    return pl.pallas_call(
        functools.partial(_basis_body, n0=n0, n1=n0, nmodes=nmodes),
        grid=grid,
        in_specs=[
            pl.BlockSpec((bm, n0), lambda i: (i, 0)),
            pl.BlockSpec((bm, n0), lambda i: (i, 0)),
            pl.BlockSpec((1, nmodes), lambda i: (0, 0)),
            pl.BlockSpec((1, nmodes), lambda i: (0, 0)),
            pl.BlockSpec((1, nmodes), lambda i: (0, 0)),
            pl.BlockSpec((1, nmodes), lambda i: (0, 0)),
        ],
        out_specs=[
            pl.BlockSpec((bm, 2 * nmodes), lambda i: (i, 0)),
            pl.BlockSpec((bm, 2 * nmodes), lambda i: (i, 0)),
        ],
        out_shape=[
            jax.ShapeDtypeStruct((n0, 2 * nmodes), jnp.float32),
            jax.ShapeDtypeStruct((n0, 2 * nmodes), jnp.float32),
        ],
    )(c0, s0, a, b, r, c)


def _recon_body(u_ref, v_ref, out_ref):
    out_ref[...] = jax.lax.dot_general(
        u_ref[...], v_ref[...],
        dimension_numbers=(((1,), (1,)), ((), ())),
        precision=jax.lax.Precision.HIGHEST,
        preferred_element_type=jnp.float32)


def _recon(u, v, bm=512, bn=512):
    n0 = u.shape[0]
    bm, bn = min(bm, n0), min(bn, n0)
    kk = u.shape[1]
    grid = (n0 // bm, n0 // bn)
    return pl.pallas_call(
        _recon_body,
        grid=grid,
        in_specs=[
            pl.BlockSpec((bm, kk), lambda i, j: (i, 0)),
            pl.BlockSpec((bn, kk), lambda i, j: (j, 0)),
        ],
        out_specs=pl.BlockSpec((bm, bn), lambda i, j: (i, j)),
        out_shape=jax.ShapeDtypeStruct((n0, n0), jnp.float32),
    )(u, v)


def _sc_topk(bmax_flat, gre2, gim2, nblocks, nkeep):
    ns = 16
    pw = nblocks // ns
    nv1 = pw // 16
    rpw = nkeep // ns
    mesh = plsc.VectorSubcoreMesh(core_axis_name="c", subcore_axis_name="s")

    def body(bmax_hbm, gre_hbm, gim_hbm, a_out, b_out, i_out,
             vals1, gidx1, res_v, res_i, merge_v, merge_i, blk_v, blk_i,
             grer, gimr, mvals, mgidx, fin_v, fin_p, outa, outb, outi,
             sh_v, sh_i, sem):
        sid = lax.axis_index("s")
        cid = lax.axis_index("c")
        lane = lax.iota(jnp.int32, 16)
        m0 = lane == 0
        neg = jnp.full((16,), -jnp.inf, jnp.float32)

        def topk_scan(vals_ref, gidx_ref, nv, out_v_ref, out_i_ref):
            def one_pass(p, _):
                def scan4(i, carry):
                    bv, bi = carry
                    for u in range(4):
                        off = (i * 4 + u) * 16
                        x = vals_ref[pl.ds(off, 16)]
                        take = x > bv
                        bv = jnp.where(take, x, bv)
                        bi = jnp.where(take, off + lane, bi)
                    return bv, bi
                bv, bi = lax.fori_loop(0, nv // 4, scan4,
                                       (neg, jnp.zeros((16,), jnp.int32)))
                dnums = lax.GatherDimensionNumbers(
                    offset_dims=(), collapsed_slice_dims=(0,),
                    start_index_map=(0,))

                def _perm(x, pm):
                    return lax.gather(
                        x, pm[:, None], dnums, slice_sizes=(1,),
                        mode=lax.GatherScatterMode.PROMISE_IN_BOUNDS)

                for s in (8, 4, 2, 1):
                    perm = (lane + s) & 15
                    vs = _perm(bv, perm)
                    is_ = _perm(bi, perm)
                    take = vs > bv
                    bv = jnp.where(take, vs, bv)
                    bi = jnp.where(take, is_, bi)
                gv = plsc.load_gather(gidx_ref, [bi])
                pos = jnp.zeros((16,), jnp.int32) + p
                plsc.store_scatter(out_v_ref, [pos], bv, mask=m0)
                plsc.store_scatter(out_i_ref, [pos], gv, mask=m0)
                plsc.store_scatter(vals_ref, [bi], neg, mask=m0)
                return 0
            lax.fori_loop(0, nkeep, one_pass, 0)

        base = sid * pw
        pltpu.sync_copy(bmax_hbm.at[pl.ds(base, pw)], vals1)

        def fill(i, _):
            gidx1[pl.ds(i * 16, 16)] = base + i * 16 + lane
            return 0
        lax.fori_loop(0, nv1, fill, 0)
        topk_scan(vals1, gidx1, nv1, res_v, res_i)

        pltpu.sync_copy(res_v, sh_v.at[pl.ds(sid * nkeep, nkeep)])
        pltpu.sync_copy(res_i, sh_i.at[pl.ds(sid * nkeep, nkeep)])
        plsc.subcore_barrier()
        pltpu.sync_copy(sh_v, merge_v)
        pltpu.sync_copy(sh_i, merge_i)
        topk_scan(merge_v, merge_i, (ns * nkeep) // 16, blk_v, blk_i)

        pltpu.async_copy(gre_hbm.at[blk_i], grer, sem).wait()
        pltpu.async_copy(gim_hbm.at[blk_i], gimr, sem).wait()
        for t in range(rpw):
            rowv = jnp.zeros((16,), jnp.int32) + (sid * rpw + t)
            for o in range(8):
                col = o * 16 + lane
                rv = plsc.load_gather(grer, [rowv, col])
                iv = plsc.load_gather(gimr, [rowv, col])
                mvals[pl.ds((t * 8 + o) * 16, 16)] = rv * rv + iv * iv
                mgidx[pl.ds((t * 8 + o) * 16, 16)] = rowv * 128 + col
        topk_scan(mvals, mgidx, rpw * 8, res_v, res_i)

        plsc.subcore_barrier()
        pltpu.sync_copy(res_v, sh_v.at[pl.ds(sid * nkeep, nkeep)])
        pltpu.sync_copy(res_i, sh_i.at[pl.ds(sid * nkeep, nkeep)])
        plsc.subcore_barrier()
        pltpu.sync_copy(sh_v, merge_v)
        pltpu.sync_copy(sh_i, merge_i)
        topk_scan(merge_v, merge_i, (ns * nkeep) // 16, fin_v, fin_p)

        @pl.when((sid == 0) & (cid == 0))
        def _():
            for g in range(nkeep // 16):
                pv = fin_p[pl.ds(g * 16, 16)]
                rowv = pv >> 7
                offv = pv & 127
                outa[pl.ds(g * 16, 16)] = plsc.load_gather(grer, [rowv, offv])
                outb[pl.ds(g * 16, 16)] = plsc.load_gather(gimr, [rowv, offv])
                outi[pl.ds(g * 16, 16)] = (
                    plsc.load_gather(blk_i, [rowv]) * 128 + offv)
            pltpu.sync_copy(outa, a_out)
            pltpu.sync_copy(outb, b_out)
            pltpu.sync_copy(outi, i_out)

    run = functools.partial(
        pl.kernel,
        mesh=mesh,
        compiler_params=pltpu.CompilerParams(needs_layout_passes=False),
        out_type=[
            jax.ShapeDtypeStruct((nkeep,), jnp.float32),
            jax.ShapeDtypeStruct((nkeep,), jnp.float32),
            jax.ShapeDtypeStruct((nkeep,), jnp.int32),
        ],
        scratch_types=[
            pltpu.VMEM((pw,), jnp.float32),
            pltpu.VMEM((pw,), jnp.int32),
            pltpu.VMEM((nkeep,), jnp.float32),
            pltpu.VMEM((nkeep,), jnp.int32),
            pltpu.VMEM((ns * nkeep,), jnp.float32),
            pltpu.VMEM((ns * nkeep,), jnp.int32),
            pltpu.VMEM((nkeep,), jnp.float32),
            pltpu.VMEM((nkeep,), jnp.int32),
            pltpu.VMEM((nkeep, 128), jnp.float32),
            pltpu.VMEM((nkeep, 128), jnp.float32),
            pltpu.VMEM((rpw * 128,), jnp.float32),
            pltpu.VMEM((rpw * 128,), jnp.int32),
            pltpu.VMEM((nkeep,), jnp.float32),
            pltpu.VMEM((nkeep,), jnp.int32),
            pltpu.VMEM((nkeep,), jnp.float32),
            pltpu.VMEM((nkeep,), jnp.float32),
            pltpu.VMEM((nkeep,), jnp.int32),
            pltpu.VMEM_SHARED((ns * nkeep,), jnp.float32),
            pltpu.VMEM_SHARED((ns * nkeep,), jnp.int32),
            pltpu.SemaphoreType.DMA,
        ],
    )(body)
    return run(bmax_flat, gre2, gim2)


N_KEEP = 64


def kernel(weight_matrix):
    n0, n1 = weight_matrix.shape
    ncp = _round_up(n1 // 2 + 1, LANE)
    if ncp > 1280:
        ncp = _round_up(n1 // 2 + 1, 2 * LANE)
    bc, bs, c0, s0 = _dft_tables(n0, n1, ncp)
    bc = jnp.asarray(bc)
    bs = jnp.asarray(bs)
    c0 = jnp.asarray(c0)
    s0 = jnp.asarray(s0)

    fre, fim = _rowfft(weight_matrix, bc, bs)
    gre, gim, bmax, fct = _colfft_ct(fre, fim)

    nblocks = n0 * (ncp // LANE)
    a, b, idx = _sc_topk(bmax.reshape(-1),
                         gre.reshape(nblocks, LANE),
                         gim.reshape(nblocks, LANE),
                         nblocks, N_KEEP)
    rp = idx // ncp
    r = (rp % fct) * fct + rp // fct
    c = idx % ncp

    u, v = _basis(c0, s0, a.reshape(1, -1), b.reshape(1, -1),
                  r.reshape(1, -1), c.reshape(1, -1))
    return _recon(u, v)


# speedup vs baseline: 11.6433x; 1.1285x over previous
import functools

import jax
import jax.numpy as jnp
import numpy as np
from jax import lax
from jax.experimental import pallas as pl
from jax.experimental.pallas import tpu as pltpu
from jax.experimental.pallas import tpu_sc as plsc

LANE = 128


def _round_up(x, m):
    return (x + m - 1) // m * m


@functools.lru_cache(maxsize=2)
def _dft_tables(n0: int, n1: int, ncp: int):
    nc = n1 // 2 + 1
    k = np.arange(ncp)
    n = np.arange(n1)
    ang1 = 2.0 * np.pi * ((np.outer(n, k) % n1) / n1)
    bc = np.cos(ang1)
    bs = np.sin(ang1)
    bc[:, nc:] = 0.0
    bs[:, nc:] = 0.0
    j = np.arange(n0)
    ang0 = 2.0 * np.pi * ((np.outer(j, j) % n0) / n0)
    c0 = np.cos(ang0).astype(np.float32)
    s0 = np.sin(ang0).astype(np.float32)
    return (bc.astype(np.float32), bs.astype(np.float32), c0, s0)


def _rowfft_body(w_ref, bc_ref, bs_ref, fre_ref, fim_ref):
    @pl.when(pl.program_id(1) == 0)
    def _():
        fre_ref[...] = jnp.zeros_like(fre_ref)
        fim_ref[...] = jnp.zeros_like(fim_ref)

    w = w_ref[...]
    fre_ref[...] += jax.lax.dot(w, bc_ref[...],
                                precision=jax.lax.Precision.HIGHEST,
                                preferred_element_type=jnp.float32)
    fim_ref[...] += jax.lax.dot(w, -bs_ref[...],
                                precision=jax.lax.Precision.HIGHEST,
                                preferred_element_type=jnp.float32)


def _rowfft(w, bc, bs, bm=512, bk=256):
    n0, n1 = w.shape
    bm, bk = min(bm, n0), min(bk, n1)
    ncp = bc.shape[1]
    grid = (n0 // bm, n1 // bk)
    return pl.pallas_call(
        _rowfft_body,
        grid=grid,
        in_specs=[
            pl.BlockSpec((bm, bk), lambda i, k: (i, k)),
            pl.BlockSpec((bk, ncp), lambda i, k: (k, 0)),
            pl.BlockSpec((bk, ncp), lambda i, k: (k, 0)),
        ],
        out_specs=[
            pl.BlockSpec((bm, ncp), lambda i, k: (i, 0)),
            pl.BlockSpec((bm, ncp), lambda i, k: (i, 0)),
        ],
        out_shape=[
            jax.ShapeDtypeStruct((n0, ncp), jnp.float32),
            jax.ShapeDtypeStruct((n0, ncp), jnp.float32),
        ],
    )(w, bc, bs)


def _transpose_body(x_ref, o_ref):
    o_ref[...] = x_ref[...].T


def _transpose(x, bm=256, bn=256):
    n0, n1 = x.shape
    bm, bn = min(bm, n0), min(bn, n1)
    return pl.pallas_call(
        _transpose_body,
        grid=(n1 // bn, n0 // bm),
        in_specs=[pl.BlockSpec((bm, bn), lambda i, j: (j, i))],
        out_specs=pl.BlockSpec((bn, bm), lambda i, j: (i, j)),
        out_shape=jax.ShapeDtypeStruct((n1, n0), x.dtype),
    )(x)


@functools.lru_cache(maxsize=2)
def _ct_tables(n0: int):
    f = int(round(np.sqrt(n0)))
    assert f * f == n0
    bqa = min(f, max(8, 256 // f))
    bqb = max(1, 256 // f)
    j1 = np.arange(f)
    r1 = np.arange(f)
    la = np.zeros((f // bqa, f * bqa, f * bqa), dtype=np.complex128)
    for g in range(f // bqa):
        for q in range(bqa):
            ang = np.outer(j1, f * r1 + g * bqa + q) * (2.0 * np.pi / n0)
            la[g, q::bqa, q::bqa] = np.exp(-1j * ang)
    j2 = np.arange(f)
    wf = np.exp(-2j * np.pi * np.outer(j2, j2) / f)
    lb = np.kron(np.eye(bqb), wf)
    return (la.real.astype(np.float32), la.imag.astype(np.float32),
            lb.real.astype(np.float32), lb.imag.astype(np.float32),
            f, bqa, bqb)


def _ct_stage_a_body(lare_ref, laim_ref, fre_ref, fim_ref, tre_ref, tim_ref):
    rows = lare_ref.shape[1]
    ncp = fre_ref.shape[2]
    la_re = lare_ref[...].reshape(rows, rows)
    la_im = laim_ref[...].reshape(rows, rows)
    f_re = fre_ref[...].reshape(rows, ncp)
    f_im = fim_ref[...].reshape(rows, ncp)
    hi = jax.lax.Precision.HIGHEST
    t_re = (jax.lax.dot(la_re, f_re, precision=hi,
                        preferred_element_type=jnp.float32)
            - jax.lax.dot(la_im, f_im, precision=hi,
                          preferred_element_type=jnp.float32))
    t_im = (jax.lax.dot(la_re, f_im, precision=hi,
                        preferred_element_type=jnp.float32)
            + jax.lax.dot(la_im, f_re, precision=hi,
                          preferred_element_type=jnp.float32))
    tre_ref[...] = t_re.reshape(tre_ref.shape)
    tim_ref[...] = t_im.reshape(tim_ref.shape)


def _ct_stage_a_real_body(lare_ref, laim_ref, w_ref, tre_ref, tim_ref):
    rows = lare_ref.shape[1]
    m = w_ref.shape[2]
    la_re = lare_ref[...].reshape(rows, rows)
    la_im = laim_ref[...].reshape(rows, rows)
    wv = w_ref[...].reshape(rows, m)
    hi = jax.lax.Precision.HIGHEST
    tre_ref[...] = jax.lax.dot(
        la_re, wv, precision=hi,
        preferred_element_type=jnp.float32).reshape(tre_ref.shape)
    tim_ref[...] = jax.lax.dot(
        la_im, wv, precision=hi,
        preferred_element_type=jnp.float32).reshape(tim_ref.shape)


def _ct_stage_b_plain_body(lbre_ref, lbim_ref, tre_ref, tim_ref,
                           zre_ref, zim_ref):
    rows = lbre_ref.shape[1]
    m = tre_ref.shape[2]
    lb_re = lbre_ref[...]
    lb_im = lbim_ref[...]
    t_re = tre_ref[...].reshape(rows, m)
    t_im = tim_ref[...].reshape(rows, m)
    hi = jax.lax.Precision.HIGHEST
    z_re = (jax.lax.dot(lb_re, t_re, precision=hi,
                        preferred_element_type=jnp.float32)
            - jax.lax.dot(lb_im, t_im, precision=hi,
                          preferred_element_type=jnp.float32))
    z_im = (jax.lax.dot(lb_re, t_im, precision=hi,
                        preferred_element_type=jnp.float32)
            + jax.lax.dot(lb_im, t_re, precision=hi,
                          preferred_element_type=jnp.float32))
    zre_ref[...] = z_re.reshape(zre_ref.shape)
    zim_ref[...] = z_im.reshape(zim_ref.shape)


@functools.lru_cache(maxsize=2)
def _ct_b_sliced(n1: int, ksl: int):
    f = int(round(np.sqrt(n1)))
    bqb = max(1, 256 // f)
    j2 = np.arange(f)
    wf = np.exp(-2j * np.pi * np.outer(j2[:ksl], j2) / f)
    lbs = np.kron(np.eye(bqb), wf)
    return lbs.real.astype(np.float32), lbs.imag.astype(np.float32), bqb


def _rowfft_ct(wt, ksl):
    n1, n0 = wt.shape
    lar, lai, _, _, f, bqa, _ = _ct_tables(n1)
    lbsr, lbsi, bqb = _ct_b_sliced(n1, ksl)
    lar, lai = jnp.asarray(lar), jnp.asarray(lai)
    lbsr, lbsi = jnp.asarray(lbsr), jnp.asarray(lbsi)
    w3 = wt.reshape(f, f, n0)
    nct = 1 if n0 <= 2048 else 2
    cta = n0 // nct
    tre, tim = pl.pallas_call(
        _ct_stage_a_real_body,
        grid=(f // bqa, nct),
        in_specs=[
            pl.BlockSpec((1, f * bqa, f * bqa), lambda g, t: (g, 0, 0)),
            pl.BlockSpec((1, f * bqa, f * bqa), lambda g, t: (g, 0, 0)),
            pl.BlockSpec((f, bqa, cta), lambda g, t: (0, g, t)),
        ],
        out_specs=[
            pl.BlockSpec((f, bqa, cta), lambda g, t: (0, g, t)),
            pl.BlockSpec((f, bqa, cta), lambda g, t: (0, g, t)),
        ],
        out_shape=[
            jax.ShapeDtypeStruct((f, f, n0), jnp.float32),
            jax.ShapeDtypeStruct((f, f, n0), jnp.float32),
        ],
    )(lar, lai, w3)
    z1re, z1im = pl.pallas_call(
        _ct_stage_b_plain_body,
        grid=(f // bqb,),
        in_specs=[
            pl.BlockSpec((bqb * ksl, bqb * f), lambda g: (0, 0)),
            pl.BlockSpec((bqb * ksl, bqb * f), lambda g: (0, 0)),
            pl.BlockSpec((bqb, f, n0), lambda g: (g, 0, 0)),
            pl.BlockSpec((bqb, f, n0), lambda g: (g, 0, 0)),
        ],
        out_specs=[
            pl.BlockSpec((bqb, ksl, n0), lambda g: (g, 0, 0)),
            pl.BlockSpec((bqb, ksl, n0), lambda g: (g, 0, 0)),
        ],
        out_shape=[
            jax.ShapeDtypeStruct((f, ksl, n0), jnp.float32),
            jax.ShapeDtypeStruct((f, ksl, n0), jnp.float32),
        ],
    )(lbsr, lbsi, tre, tim)
    return z1re.reshape(f * ksl, n0), z1im.reshape(f * ksl, n0), f


def _ct_stage_b_body(lbre_ref, lbim_ref, mask_ref, tre_ref, tim_ref,
                     zre_ref, zim_ref, bmax_ref, *, nblk):
    rows = lbre_ref.shape[0]
    ncp = tre_ref.shape[2]
    lb_re = lbre_ref[...]
    lb_im = lbim_ref[...]
    t_re = tre_ref[...].reshape(rows, ncp)
    t_im = tim_ref[...].reshape(rows, ncp)
    hi = jax.lax.Precision.HIGHEST
    z_re = (jax.lax.dot(lb_re, t_re, precision=hi,
                        preferred_element_type=jnp.float32)
            - jax.lax.dot(lb_im, t_im, precision=hi,
                          preferred_element_type=jnp.float32))
    z_im = (jax.lax.dot(lb_re, t_im, precision=hi,
                        preferred_element_type=jnp.float32)
            + jax.lax.dot(lb_im, t_re, precision=hi,
                          preferred_element_type=jnp.float32))
    mag2 = (z_re * z_re + z_im * z_im) * mask_ref[...]
    for j in range(nblk):
        bmax_ref[:, j] = jnp.max(mag2[:, j * LANE:(j + 1) * LANE], axis=1)
    zre_ref[...] = z_re.reshape(zre_ref.shape)
    zim_ref[...] = z_im.reshape(zim_ref.shape)


def _colfft_ct(fre, fim, mask):
    n0, ncp = fre.shape
    lar, lai, lbr, lbi, f, bqa, bqb = _ct_tables(n0)
    lar, lai = jnp.asarray(lar), jnp.asarray(lai)
    lbr, lbi = jnp.asarray(lbr), jnp.asarray(lbi)
    nblk = ncp // LANE
    f3 = (f, f, ncp)
    fre3 = fre.reshape(f3)
    fim3 = fim.reshape(f3)
    nct = 1 if ncp <= 1280 else 2
    cta = ncp // nct
    tre, tim = pl.pallas_call(
        _ct_stage_a_body,
        grid=(f // bqa, nct),
        in_specs=[
            pl.BlockSpec((1, f * bqa, f * bqa), lambda g, t: (g, 0, 0)),
            pl.BlockSpec((1, f * bqa, f * bqa), lambda g, t: (g, 0, 0)),
            pl.BlockSpec((f, bqa, cta), lambda g, t: (0, g, t)),
            pl.BlockSpec((f, bqa, cta), lambda g, t: (0, g, t)),
        ],
        out_specs=[
            pl.BlockSpec((f, bqa, cta), lambda g, t: (0, g, t)),
            pl.BlockSpec((f, bqa, cta), lambda g, t: (0, g, t)),
        ],
        out_shape=[
            jax.ShapeDtypeStruct(f3, jnp.float32),
            jax.ShapeDtypeStruct(f3, jnp.float32),
        ],
    )(lar, lai, fre3, fim3)
    zre, zim, bmax = pl.pallas_call(
        functools.partial(_ct_stage_b_body, nblk=nblk),
        grid=(f // bqb,),
        in_specs=[
            pl.BlockSpec((f * bqb, f * bqb), lambda g: (0, 0)),
            pl.BlockSpec((f * bqb, f * bqb), lambda g: (0, 0)),
            pl.BlockSpec((1, ncp), lambda g: (0, 0)),
            pl.BlockSpec((bqb, f, ncp), lambda g: (g, 0, 0)),
            pl.BlockSpec((bqb, f, ncp), lambda g: (g, 0, 0)),
        ],
        out_specs=[
            pl.BlockSpec((bqb, f, ncp), lambda g: (g, 0, 0)),
            pl.BlockSpec((bqb, f, ncp), lambda g: (g, 0, 0)),
            pl.BlockSpec((f * bqb, nblk), lambda g: (g, 0)),
        ],
        out_shape=[
            jax.ShapeDtypeStruct(f3, jnp.float32),
            jax.ShapeDtypeStruct(f3, jnp.float32),
            jax.ShapeDtypeStruct((n0, nblk), jnp.float32),
        ],
    )(lbr, lbi, mask, tre, tim)
    return (zre.reshape(n0, ncp), zim.reshape(n0, ncp), bmax, f)


def _colfft_body(c0_ref, s0_ref, fre_ref, fim_ref,
                 gre_ref, gim_ref, bmax_ref, *, nblk):
    @pl.when(pl.program_id(1) == 0)
    def _():
        gre_ref[...] = jnp.zeros_like(gre_ref)
        gim_ref[...] = jnp.zeros_like(gim_ref)

    c0 = c0_ref[...]
    s0 = s0_ref[...]
    fre = fre_ref[...]
    fim = fim_ref[...]
    hi = jax.lax.Precision.HIGHEST
    gre_ref[...] += (jax.lax.dot(c0, fre, precision=hi,
                                 preferred_element_type=jnp.float32)
                     + jax.lax.dot(s0, fim, precision=hi,
                                   preferred_element_type=jnp.float32))
    gim_ref[...] += (jax.lax.dot(c0, fim, precision=hi,
                                 preferred_element_type=jnp.float32)
                     - jax.lax.dot(s0, fre, precision=hi,
                                   preferred_element_type=jnp.float32))

    @pl.when(pl.program_id(1) == pl.num_programs(1) - 1)
    def _():
        gre = gre_ref[...]
        gim = gim_ref[...]
        mag2 = gre * gre + gim * gim
        for j in range(nblk):
            blk = mag2[:, j * LANE:(j + 1) * LANE]
            bmax_ref[:, j] = jnp.max(blk, axis=1)


def _colfft(c0, s0, fre, fim, bm=256, bk=256):
    n0 = c0.shape[0]
    bm, bk = min(bm, n0), min(bk, n0)
    ncp = fre.shape[1]
    nblk = ncp // LANE
    grid = (n0 // bm, n0 // bk)
    return pl.pallas_call(
        functools.partial(_colfft_body, nblk=nblk),
        grid=grid,
        in_specs=[
            pl.BlockSpec((bm, bk), lambda i, k: (i, k)),
            pl.BlockSpec((bm, bk), lambda i, k: (i, k)),
            pl.BlockSpec((bk, ncp), lambda i, k: (k, 0)),
            pl.BlockSpec((bk, ncp), lambda i, k: (k, 0)),
        ],
        out_specs=[
            pl.BlockSpec((bm, ncp), lambda i, k: (i, 0)),
            pl.BlockSpec((bm, ncp), lambda i, k: (i, 0)),
            pl.BlockSpec((bm, nblk), lambda i, k: (i, 0)),
        ],
        out_shape=[
            jax.ShapeDtypeStruct((n0, ncp), jnp.float32),
            jax.ShapeDtypeStruct((n0, ncp), jnp.float32),
            jax.ShapeDtypeStruct((n0, nblk), jnp.float32),
        ],
    )(c0, s0, fre, fim)


def _basis_body(c0_ref, s0_ref, a_ref, b_ref, r_ref, c_ref,
                u_ref, v_ref, *, n0, n1, nmodes):
    rows = jax.lax.broadcasted_iota(jnp.int32, (c0_ref.shape[0], 1), 0) \
        + pl.program_id(0) * c0_ref.shape[0]
    r = r_ref[0, :]
    c = c_ref[0, :]
    a = a_ref[0, :]
    b = b_ref[0, :]
    onehot_r = (jax.lax.broadcasted_iota(jnp.int32, (c0_ref.shape[1], nmodes), 0)
                == r[None, :]).astype(jnp.float32)
    onehot_c = (jax.lax.broadcasted_iota(jnp.int32, (c0_ref.shape[1], nmodes), 0)
                == c[None, :]).astype(jnp.bfloat16)
    onehot_r = onehot_r.astype(jnp.bfloat16)

    def _gdot(x, oh):
        xh = x.astype(jnp.bfloat16)
        xl = (x - xh.astype(jnp.float32)).astype(jnp.bfloat16)
        return (jax.lax.dot(xh, oh, preferred_element_type=jnp.float32)
                + jax.lax.dot(xl, oh, preferred_element_type=jnp.float32))

    cr = _gdot(c0_ref[...], onehot_r)
    sr = _gdot(s0_ref[...], onehot_r)
    cc = _gdot(c0_ref[...], onehot_c)
    sc = _gdot(s0_ref[...], onehot_c)
    del rows
    p = a[None, :] * cr - b[None, :] * sr
    q = -(a[None, :] * sr + b[None, :] * cr)
    u_ref[...] = jnp.concatenate([p, q], axis=1)
    w = jnp.where((c == 0) | (c == n1 // 2), 1.0, 2.0) / (
        jnp.float32(n0) * jnp.float32(n1))
    v_ref[...] = jnp.concatenate([w[None, :] * cc, w[None, :] * sc], axis=1)


def _basis(c0, s0, a, b, r, c, bm=256):
    n0 = c0.shape[0]
    bm = min(bm, n0)
    nmodes = a.shape[1]
    grid = (n0 // bm,)
    return pl.pallas_call(
        functools.partial(_basis_body, n0=n0, n1=n0, nmodes=nmodes),
        grid=grid,
        in_specs=[
            pl.BlockSpec((bm, n0), lambda i: (i, 0)),
            pl.BlockSpec((bm, n0), lambda i: (i, 0)),
            pl.BlockSpec((1, nmodes), lambda i: (0, 0)),
            pl.BlockSpec((1, nmodes), lambda i: (0, 0)),
            pl.BlockSpec((1, nmodes), lambda i: (0, 0)),
            pl.BlockSpec((1, nmodes), lambda i: (0, 0)),
        ],
        out_specs=[
            pl.BlockSpec((bm, 2 * nmodes), lambda i: (i, 0)),
            pl.BlockSpec((bm, 2 * nmodes), lambda i: (i, 0)),
        ],
        out_shape=[
            jax.ShapeDtypeStruct((n0, 2 * nmodes), jnp.float32),
            jax.ShapeDtypeStruct((n0, 2 * nmodes), jnp.float32),
        ],
    )(c0, s0, a, b, r, c)


def _recon_body(u_ref, v_ref, out_ref):
    out_ref[...] = jax.lax.dot_general(
        u_ref[...], v_ref[...],
        dimension_numbers=(((1,), (1,)), ((), ())),
        precision=jax.lax.Precision.HIGHEST,
        preferred_element_type=jnp.float32)


def _recon(u, v, bm=512, bn=512):
    n0 = u.shape[0]
    bm, bn = min(bm, n0), min(bn, n0)
    kk = u.shape[1]
    grid = (n0 // bm, n0 // bn)
    return pl.pallas_call(
        _recon_body,
        grid=grid,
        in_specs=[
            pl.BlockSpec((bm, kk), lambda i, j: (i, 0)),
            pl.BlockSpec((bn, kk), lambda i, j: (j, 0)),
        ],
        out_specs=pl.BlockSpec((bm, bn), lambda i, j: (i, j)),
        out_shape=jax.ShapeDtypeStruct((n0, n0), jnp.float32),
    )(u, v)


def _sc_topk(bmax_flat, gre2, gim2, nblocks, nkeep):
    ns = 16
    pw = nblocks // ns
    nv1 = pw // 16
    rpw = nkeep // ns
    mesh = plsc.VectorSubcoreMesh(core_axis_name="c", subcore_axis_name="s")

    def body(bmax_hbm, gre_hbm, gim_hbm, a_out, b_out, i_out,
             vals1, gidx1, res_v, res_i, merge_v, merge_i, blk_v, blk_i,
             grer, gimr, mvals, mgidx, fin_v, fin_p, outa, outb, outi,
             sh_v, sh_i, sem):
        sid = lax.axis_index("s")
        cid = lax.axis_index("c")
        lane = lax.iota(jnp.int32, 16)
        m0 = lane == 0
        neg = jnp.full((16,), -jnp.inf, jnp.float32)

        def topk_scan(vals_ref, gidx_ref, nv, out_v_ref, out_i_ref):
            def one_pass(p, _):
                def scan4(i, carry):
                    bv, bi = carry
                    for u in range(4):
                        off = (i * 4 + u) * 16
                        x = vals_ref[pl.ds(off, 16)]
                        take = x > bv
                        bv = jnp.where(take, x, bv)
                        bi = jnp.where(take, off + lane, bi)
                    return bv, bi
                bv, bi = lax.fori_loop(0, nv // 4, scan4,
                                       (neg, jnp.zeros((16,), jnp.int32)))
                dnums = lax.GatherDimensionNumbers(
                    offset_dims=(), collapsed_slice_dims=(0,),
                    start_index_map=(0,))

                def _perm(x, pm):
                    return lax.gather(
                        x, pm[:, None], dnums, slice_sizes=(1,),
                        mode=lax.GatherScatterMode.PROMISE_IN_BOUNDS)

                for s in (8, 4, 2, 1):
                    perm = (lane + s) & 15
                    vs = _perm(bv, perm)
                    is_ = _perm(bi, perm)
                    take = vs > bv
                    bv = jnp.where(take, vs, bv)
                    bi = jnp.where(take, is_, bi)
                gv = plsc.load_gather(gidx_ref, [bi])
                pos = jnp.zeros((16,), jnp.int32) + p
                plsc.store_scatter(out_v_ref, [pos], bv, mask=m0)
                plsc.store_scatter(out_i_ref, [pos], gv, mask=m0)
                plsc.store_scatter(vals_ref, [bi], neg, mask=m0)
                return 0
            lax.fori_loop(0, nkeep, one_pass, 0)

        base = sid * pw
        pltpu.sync_copy(bmax_hbm.at[pl.ds(base, pw)], vals1)

        def fill(i, _):
            gidx1[pl.ds(i * 16, 16)] = base + i * 16 + lane
            return 0
        lax.fori_loop(0, nv1, fill, 0)
        topk_scan(vals1, gidx1, nv1, res_v, res_i)

        pltpu.sync_copy(res_v, sh_v.at[pl.ds(sid * nkeep, nkeep)])
        pltpu.sync_copy(res_i, sh_i.at[pl.ds(sid * nkeep, nkeep)])
        plsc.subcore_barrier()
        pltpu.sync_copy(sh_v, merge_v)
        pltpu.sync_copy(sh_i, merge_i)
        topk_scan(merge_v, merge_i, (ns * nkeep) // 16, blk_v, blk_i)

        pltpu.async_copy(gre_hbm.at[blk_i], grer, sem).wait()
        pltpu.async_copy(gim_hbm.at[blk_i], gimr, sem).wait()
        for t in range(rpw):
            rowv = jnp.zeros((16,), jnp.int32) + (sid * rpw + t)
            for o in range(8):
                col = o * 16 + lane
                rv = plsc.load_gather(grer, [rowv, col])
                iv = plsc.load_gather(gimr, [rowv, col])
                mvals[pl.ds((t * 8 + o) * 16, 16)] = rv * rv + iv * iv
                mgidx[pl.ds((t * 8 + o) * 16, 16)] = rowv * 128 + col
        topk_scan(mvals, mgidx, rpw * 8, res_v, res_i)

        plsc.subcore_barrier()
        pltpu.sync_copy(res_v, sh_v.at[pl.ds(sid * nkeep, nkeep)])
        pltpu.sync_copy(res_i, sh_i.at[pl.ds(sid * nkeep, nkeep)])
        plsc.subcore_barrier()
        pltpu.sync_copy(sh_v, merge_v)
        pltpu.sync_copy(sh_i, merge_i)
        topk_scan(merge_v, merge_i, (ns * nkeep) // 16, fin_v, fin_p)

        @pl.when((sid == 0) & (cid == 0))
        def _():
            for g in range(nkeep // 16):
                pv = fin_p[pl.ds(g * 16, 16)]
                rowv = pv >> 7
                offv = pv & 127
                outa[pl.ds(g * 16, 16)] = plsc.load_gather(grer, [rowv, offv])
                outb[pl.ds(g * 16, 16)] = plsc.load_gather(gimr, [rowv, offv])
                outi[pl.ds(g * 16, 16)] = (
                    plsc.load_gather(blk_i, [rowv]) * 128 + offv)
            pltpu.sync_copy(outa, a_out)
            pltpu.sync_copy(outb, b_out)
            pltpu.sync_copy(outi, i_out)

    run = functools.partial(
        pl.kernel,
        mesh=mesh,
        compiler_params=pltpu.CompilerParams(needs_layout_passes=False),
        out_type=[
            jax.ShapeDtypeStruct((nkeep,), jnp.float32),
            jax.ShapeDtypeStruct((nkeep,), jnp.float32),
            jax.ShapeDtypeStruct((nkeep,), jnp.int32),
        ],
        scratch_types=[
            pltpu.VMEM((pw,), jnp.float32),
            pltpu.VMEM((pw,), jnp.int32),
            pltpu.VMEM((nkeep,), jnp.float32),
            pltpu.VMEM((nkeep,), jnp.int32),
            pltpu.VMEM((ns * nkeep,), jnp.float32),
            pltpu.VMEM((ns * nkeep,), jnp.int32),
            pltpu.VMEM((nkeep,), jnp.float32),
            pltpu.VMEM((nkeep,), jnp.int32),
            pltpu.VMEM((nkeep, 128), jnp.float32),
            pltpu.VMEM((nkeep, 128), jnp.float32),
            pltpu.VMEM((rpw * 128,), jnp.float32),
            pltpu.VMEM((rpw * 128,), jnp.int32),
            pltpu.VMEM((nkeep,), jnp.float32),
            pltpu.VMEM((nkeep,), jnp.int32),
            pltpu.VMEM((nkeep,), jnp.float32),
            pltpu.VMEM((nkeep,), jnp.float32),
            pltpu.VMEM((nkeep,), jnp.int32),
            pltpu.VMEM_SHARED((ns * nkeep,), jnp.float32),
            pltpu.VMEM_SHARED((ns * nkeep,), jnp.int32),
            pltpu.SemaphoreType.DMA,
        ],
    )(body)
    return run(bmax_flat, gre2, gim2)


N_KEEP = 64


def kernel(weight_matrix):
    n0, n1 = weight_matrix.shape
    f1 = int(round(np.sqrt(n1)))
    ksl = n1 // 2 // f1 + 1
    align = 2 * LANE if n1 > 2048 else LANE
    while (f1 * ksl) % align:
        ksl += 1
    ncp = f1 * ksl
    _, _, c0, s0 = _dft_tables(n0, n1, ncp)
    c0 = jnp.asarray(c0)
    s0 = jnp.asarray(s0)

    cs_np = np.arange(ncp)
    truec_np = f1 * (cs_np % ksl) + cs_np // ksl
    mask = jnp.asarray((truec_np <= n1 // 2)
                       .astype(np.float32).reshape(1, ncp))

    wt = _transpose(weight_matrix)
    z1re, z1im, _ = _rowfft_ct(wt, ksl)
    f2re = _transpose(z1re)
    f2im = _transpose(z1im)
    gre, gim, bmax, f0 = _colfft_ct(f2re, f2im, mask)

    nblocks = n0 * (ncp // LANE)
    a, b, idx = _sc_topk(bmax.reshape(-1),
                         gre.reshape(nblocks, LANE),
                         gim.reshape(nblocks, LANE),
                         nblocks, N_KEEP)
    rp = idx // ncp
    r = (rp % f0) * f0 + rp // f0
    cp = idx % ncp
    c = f1 * (cp % ksl) + cp // ksl

    u, v = _basis(c0, s0, a.reshape(1, -1), b.reshape(1, -1),
                  r.reshape(1, -1), c.reshape(1, -1))
    return _recon(u, v)


# speedup vs baseline: 11.6573x; 1.0012x over previous
import functools

import jax
import jax.numpy as jnp
import numpy as np
from jax import lax
from jax.experimental import pallas as pl
from jax.experimental.pallas import tpu as pltpu
from jax.experimental.pallas import tpu_sc as plsc

LANE = 128


def _round_up(x, m):
    return (x + m - 1) // m * m


@functools.lru_cache(maxsize=2)
def _gather_tables(n: int):
    j = np.arange(n)
    ang = 2.0 * np.pi * ((np.outer(j, j) % n) / n)
    return np.cos(ang).astype(np.float32), np.sin(ang).astype(np.float32)


def _transpose_body(x_ref, o_ref):
    o_ref[...] = x_ref[...].T


def _transpose(x, bm=256, bn=256):
    n0, n1 = x.shape
    bm, bn = min(bm, n0), min(bn, n1)
    return pl.pallas_call(
        _transpose_body,
        grid=(n1 // bn, n0 // bm),
        in_specs=[pl.BlockSpec((bm, bn), lambda i, j: (j, i))],
        out_specs=pl.BlockSpec((bn, bm), lambda i, j: (i, j)),
        out_shape=jax.ShapeDtypeStruct((n1, n0), x.dtype),
    )(x)


@functools.lru_cache(maxsize=2)
def _ct_tables(n0: int):
    f = int(round(np.sqrt(n0)))
    assert f * f == n0
    bqa = min(f, max(8, 256 // f))
    bqb = max(1, 256 // f)
    j1 = np.arange(f)
    r1 = np.arange(f)
    la = np.zeros((f // bqa, f * bqa, f * bqa), dtype=np.complex128)
    for g in range(f // bqa):
        for q in range(bqa):
            ang = np.outer(j1, f * r1 + g * bqa + q) * (2.0 * np.pi / n0)
            la[g, q::bqa, q::bqa] = np.exp(-1j * ang)
    j2 = np.arange(f)
    wf = np.exp(-2j * np.pi * np.outer(j2, j2) / f)
    lb = np.kron(np.eye(bqb), wf)
    return (la.real.astype(np.float32), la.imag.astype(np.float32),
            lb.real.astype(np.float32), lb.imag.astype(np.float32),
            f, bqa, bqb)


def _ct_stage_a_body(lare_ref, laim_ref, fre_ref, fim_ref, tre_ref, tim_ref):
    rows = lare_ref.shape[1]
    ncp = fre_ref.shape[2]
    la_re = lare_ref[...].reshape(rows, rows)
    la_im = laim_ref[...].reshape(rows, rows)
    f_re = fre_ref[...].reshape(rows, ncp)
    f_im = fim_ref[...].reshape(rows, ncp)
    hi = jax.lax.Precision.HIGHEST
    t_re = (jax.lax.dot(la_re, f_re, precision=hi,
                        preferred_element_type=jnp.float32)
            - jax.lax.dot(la_im, f_im, precision=hi,
                          preferred_element_type=jnp.float32))
    t_im = (jax.lax.dot(la_re, f_im, precision=hi,
                        preferred_element_type=jnp.float32)
            + jax.lax.dot(la_im, f_re, precision=hi,
                          preferred_element_type=jnp.float32))
    tre_ref[...] = t_re.reshape(tre_ref.shape)
    tim_ref[...] = t_im.reshape(tim_ref.shape)


def _ct_stage_a_real_body(lare_ref, laim_ref, w_ref, tre_ref, tim_ref):
    rows = lare_ref.shape[1]
    m = w_ref.shape[2]
    la_re = lare_ref[...].reshape(rows, rows)
    la_im = laim_ref[...].reshape(rows, rows)
    wv = w_ref[...].reshape(rows, m)
    hi = jax.lax.Precision.HIGHEST
    tre_ref[...] = jax.lax.dot(
        la_re, wv, precision=hi,
        preferred_element_type=jnp.float32).reshape(tre_ref.shape)
    tim_ref[...] = jax.lax.dot(
        la_im, wv, precision=hi,
        preferred_element_type=jnp.float32).reshape(tim_ref.shape)


def _ct_stage_b_plain_body(lbre_ref, lbim_ref, tre_ref, tim_ref,
                           zre_ref, zim_ref):
    rows = lbre_ref.shape[1]
    m = tre_ref.shape[2]
    lb_re = lbre_ref[...]
    lb_im = lbim_ref[...]
    t_re = tre_ref[...].reshape(rows, m)
    t_im = tim_ref[...].reshape(rows, m)
    hi = jax.lax.Precision.HIGHEST
    z_re = (jax.lax.dot(lb_re, t_re, precision=hi,
                        preferred_element_type=jnp.float32)
            - jax.lax.dot(lb_im, t_im, precision=hi,
                          preferred_element_type=jnp.float32))
    z_im = (jax.lax.dot(lb_re, t_im, precision=hi,
                        preferred_element_type=jnp.float32)
            + jax.lax.dot(lb_im, t_re, precision=hi,
                          preferred_element_type=jnp.float32))
    zre_ref[...] = z_re.reshape(zre_ref.shape)
    zim_ref[...] = z_im.reshape(zim_ref.shape)


@functools.lru_cache(maxsize=2)
def _ct_b_sliced(n1: int, ksl: int):
    f = int(round(np.sqrt(n1)))
    bqb = max(1, 256 // f)
    j2 = np.arange(f)
    wf = np.exp(-2j * np.pi * np.outer(j2[:ksl], j2) / f)
    lbs = np.kron(np.eye(bqb), wf)
    return lbs.real.astype(np.float32), lbs.imag.astype(np.float32), bqb


def _rowfft_ct(wt, ksl):
    n1, n0 = wt.shape
    lar, lai, _, _, f, bqa, _ = _ct_tables(n1)
    lbsr, lbsi, bqb = _ct_b_sliced(n1, ksl)
    lar, lai = jnp.asarray(lar), jnp.asarray(lai)
    lbsr, lbsi = jnp.asarray(lbsr), jnp.asarray(lbsi)
    w3 = wt.reshape(f, f, n0)
    nct = 1 if n0 <= 2048 else 2
    cta = n0 // nct
    tre, tim = pl.pallas_call(
        _ct_stage_a_real_body,
        grid=(f // bqa, nct),
        in_specs=[
            pl.BlockSpec((1, f * bqa, f * bqa), lambda g, t: (g, 0, 0)),
            pl.BlockSpec((1, f * bqa, f * bqa), lambda g, t: (g, 0, 0)),
            pl.BlockSpec((f, bqa, cta), lambda g, t: (0, g, t)),
        ],
        out_specs=[
            pl.BlockSpec((f, bqa, cta), lambda g, t: (0, g, t)),
            pl.BlockSpec((f, bqa, cta), lambda g, t: (0, g, t)),
        ],
        out_shape=[
            jax.ShapeDtypeStruct((f, f, n0), jnp.float32),
            jax.ShapeDtypeStruct((f, f, n0), jnp.float32),
        ],
    )(lar, lai, w3)
    z1re, z1im = pl.pallas_call(
        _ct_stage_b_plain_body,
        grid=(f // bqb,),
        in_specs=[
            pl.BlockSpec((bqb * ksl, bqb * f), lambda g: (0, 0)),
            pl.BlockSpec((bqb * ksl, bqb * f), lambda g: (0, 0)),
            pl.BlockSpec((bqb, f, n0), lambda g: (g, 0, 0)),
            pl.BlockSpec((bqb, f, n0), lambda g: (g, 0, 0)),
        ],
        out_specs=[
            pl.BlockSpec((bqb, ksl, n0), lambda g: (g, 0, 0)),
            pl.BlockSpec((bqb, ksl, n0), lambda g: (g, 0, 0)),
        ],
        out_shape=[
            jax.ShapeDtypeStruct((f, ksl, n0), jnp.float32),
            jax.ShapeDtypeStruct((f, ksl, n0), jnp.float32),
        ],
    )(lbsr, lbsi, tre, tim)
    return z1re.reshape(f * ksl, n0), z1im.reshape(f * ksl, n0), f


def _ct_stage_b_body(lbre_ref, lbim_ref, mask_ref, tre_ref, tim_ref,
                     zre_ref, zim_ref, bmax_ref, *, nblk):
    rows = lbre_ref.shape[0]
    ncp = tre_ref.shape[2]
    lb_re = lbre_ref[...]
    lb_im = lbim_ref[...]
    t_re = tre_ref[...].reshape(rows, ncp)
    t_im = tim_ref[...].reshape(rows, ncp)
    hi = jax.lax.Precision.HIGHEST
    z_re = (jax.lax.dot(lb_re, t_re, precision=hi,
                        preferred_element_type=jnp.float32)
            - jax.lax.dot(lb_im, t_im, precision=hi,
                          preferred_element_type=jnp.float32))
    z_im = (jax.lax.dot(lb_re, t_im, precision=hi,
                        preferred_element_type=jnp.float32)
            + jax.lax.dot(lb_im, t_re, precision=hi,
                          preferred_element_type=jnp.float32))
    mag2 = (z_re * z_re + z_im * z_im) * mask_ref[...]
    for j in range(nblk):
        bmax_ref[:, j] = jnp.max(mag2[:, j * LANE:(j + 1) * LANE], axis=1)
    zre_ref[...] = z_re.reshape(zre_ref.shape)
    zim_ref[...] = z_im.reshape(zim_ref.shape)


def _colfft_ct(fre, fim, mask):
    n0, ncp = fre.shape
    lar, lai, lbr, lbi, f, bqa, bqb = _ct_tables(n0)
    lar, lai = jnp.asarray(lar), jnp.asarray(lai)
    lbr, lbi = jnp.asarray(lbr), jnp.asarray(lbi)
    nblk = ncp // LANE
    f3 = (f, f, ncp)
    fre3 = fre.reshape(f3)
    fim3 = fim.reshape(f3)
    nct = 1 if ncp <= 1280 else 2
    cta = ncp // nct
    tre, tim = pl.pallas_call(
        _ct_stage_a_body,
        grid=(f // bqa, nct),
        in_specs=[
            pl.BlockSpec((1, f * bqa, f * bqa), lambda g, t: (g, 0, 0)),
            pl.BlockSpec((1, f * bqa, f * bqa), lambda g, t: (g, 0, 0)),
            pl.BlockSpec((f, bqa, cta), lambda g, t: (0, g, t)),
            pl.BlockSpec((f, bqa, cta), lambda g, t: (0, g, t)),
        ],
        out_specs=[
            pl.BlockSpec((f, bqa, cta), lambda g, t: (0, g, t)),
            pl.BlockSpec((f, bqa, cta), lambda g, t: (0, g, t)),
        ],
        out_shape=[
            jax.ShapeDtypeStruct(f3, jnp.float32),
            jax.ShapeDtypeStruct(f3, jnp.float32),
        ],
    )(lar, lai, fre3, fim3)
    zre, zim, bmax = pl.pallas_call(
        functools.partial(_ct_stage_b_body, nblk=nblk),
        grid=(f // bqb,),
        in_specs=[
            pl.BlockSpec((f * bqb, f * bqb), lambda g: (0, 0)),
            pl.BlockSpec((f * bqb, f * bqb), lambda g: (0, 0)),
            pl.BlockSpec((1, ncp), lambda g: (0, 0)),
            pl.BlockSpec((bqb, f, ncp), lambda g: (g, 0, 0)),
            pl.BlockSpec((bqb, f, ncp), lambda g: (g, 0, 0)),
        ],
        out_specs=[
            pl.BlockSpec((bqb, f, ncp), lambda g: (g, 0, 0)),
            pl.BlockSpec((bqb, f, ncp), lambda g: (g, 0, 0)),
            pl.BlockSpec((f * bqb, nblk), lambda g: (g, 0)),
        ],
        out_shape=[
            jax.ShapeDtypeStruct(f3, jnp.float32),
            jax.ShapeDtypeStruct(f3, jnp.float32),
            jax.ShapeDtypeStruct((n0, nblk), jnp.float32),
        ],
    )(lbr, lbi, mask, tre, tim)
    return (zre.reshape(n0, ncp), zim.reshape(n0, ncp), bmax, f)


def _colfft_body(c0_ref, s0_ref, fre_ref, fim_ref,
                 gre_ref, gim_ref, bmax_ref, *, nblk):
    @pl.when(pl.program_id(1) == 0)
    def _():
        gre_ref[...] = jnp.zeros_like(gre_ref)
        gim_ref[...] = jnp.zeros_like(gim_ref)

    c0 = c0_ref[...]
    s0 = s0_ref[...]
    fre = fre_ref[...]
    fim = fim_ref[...]
    hi = jax.lax.Precision.HIGHEST
    gre_ref[...] += (jax.lax.dot(c0, fre, precision=hi,
                                 preferred_element_type=jnp.float32)
                     + jax.lax.dot(s0, fim, precision=hi,
                                   preferred_element_type=jnp.float32))
    gim_ref[...] += (jax.lax.dot(c0, fim, precision=hi,
                                 preferred_element_type=jnp.float32)
                     - jax.lax.dot(s0, fre, precision=hi,
                                   preferred_element_type=jnp.float32))

    @pl.when(pl.program_id(1) == pl.num_programs(1) - 1)
    def _():
        gre = gre_ref[...]
        gim = gim_ref[...]
        mag2 = gre * gre + gim * gim
        for j in range(nblk):
            blk = mag2[:, j * LANE:(j + 1) * LANE]
            bmax_ref[:, j] = jnp.max(blk, axis=1)


def _colfft(c0, s0, fre, fim, bm=256, bk=256):
    n0 = c0.shape[0]
    bm, bk = min(bm, n0), min(bk, n0)
    ncp = fre.shape[1]
    nblk = ncp // LANE
    grid = (n0 // bm, n0 // bk)
    return pl.pallas_call(
        functools.partial(_colfft_body, nblk=nblk),
        grid=grid,
        in_specs=[
            pl.BlockSpec((bm, bk), lambda i, k: (i, k)),
            pl.BlockSpec((bm, bk), lambda i, k: (i, k)),
            pl.BlockSpec((bk, ncp), lambda i, k: (k, 0)),
            pl.BlockSpec((bk, ncp), lambda i, k: (k, 0)),
        ],
        out_specs=[
            pl.BlockSpec((bm, ncp), lambda i, k: (i, 0)),
            pl.BlockSpec((bm, ncp), lambda i, k: (i, 0)),
            pl.BlockSpec((bm, nblk), lambda i, k: (i, 0)),
        ],
        out_shape=[
            jax.ShapeDtypeStruct((n0, ncp), jnp.float32),
            jax.ShapeDtypeStruct((n0, ncp), jnp.float32),
            jax.ShapeDtypeStruct((n0, nblk), jnp.float32),
        ],
    )(c0, s0, fre, fim)


def _basis_body(c0_ref, s0_ref, a_ref, b_ref, r_ref, c_ref,
                u_ref, v_ref, *, n0, n1, nmodes):
    r = r_ref[0, :]
    c = c_ref[0, :]
    a = a_ref[0, :]
    b = b_ref[0, :]
    onehot_r = (jax.lax.broadcasted_iota(jnp.int32, (c0_ref.shape[1], nmodes), 0)
                == r[None, :]).astype(jnp.float32)
    onehot_c = (jax.lax.broadcasted_iota(jnp.int32, (c0_ref.shape[1], nmodes), 0)
                == c[None, :]).astype(jnp.bfloat16)
    onehot_r = onehot_r.astype(jnp.bfloat16)

    def _gdot(x, oh):
        xh = x.astype(jnp.bfloat16)
        xl = (x - xh.astype(jnp.float32)).astype(jnp.bfloat16)
        return (jax.lax.dot(xh, oh, preferred_element_type=jnp.float32)
                + jax.lax.dot(xl, oh, preferred_element_type=jnp.float32))

    cr = _gdot(c0_ref[...], onehot_r)
    sr = _gdot(s0_ref[...], onehot_r)
    cc = _gdot(c0_ref[...], onehot_c)
    sc = _gdot(s0_ref[...], onehot_c)
    p = a[None, :] * cr - b[None, :] * sr
    q = -(a[None, :] * sr + b[None, :] * cr)
    u_ref[...] = jnp.concatenate([p, q], axis=1)
    w = jnp.where((c == 0) | (c == n1 // 2), 1.0, 2.0) / (
        jnp.float32(n0) * jnp.float32(n1))
    v_ref[...] = jnp.concatenate([w[None, :] * cc, w[None, :] * sc], axis=1)


def _basis(c0, s0, a, b, r, c, bm=256):
    n0 = c0.shape[0]
    bm = min(bm, n0)
    nmodes = a.shape[1]
    grid = (n0 // bm,)
    return pl.pallas_call(
        functools.partial(_basis_body, n0=n0, n1=n0, nmodes=nmodes),
        grid=grid,
        in_specs=[
            pl.BlockSpec((bm, n0), lambda i: (i, 0)),
            pl.BlockSpec((bm, n0), lambda i: (i, 0)),
            pl.BlockSpec((1, nmodes), lambda i: (0, 0)),
            pl.BlockSpec((1, nmodes), lambda i: (0, 0)),
            pl.BlockSpec((1, nmodes), lambda i: (0, 0)),
            pl.BlockSpec((1, nmodes), lambda i: (0, 0)),
        ],
        out_specs=[
            pl.BlockSpec((bm, 2 * nmodes), lambda i: (i, 0)),
            pl.BlockSpec((bm, 2 * nmodes), lambda i: (i, 0)),
        ],
        out_shape=[
            jax.ShapeDtypeStruct((n0, 2 * nmodes), jnp.float32),
            jax.ShapeDtypeStruct((n0, 2 * nmodes), jnp.float32),
        ],
    )(c0, s0, a, b, r, c)


def _recon_body(u_ref, v_ref, out_ref):
    out_ref[...] = jax.lax.dot_general(
        u_ref[...], v_ref[...],
        dimension_numbers=(((1,), (1,)), ((), ())),
        precision=jax.lax.Precision.HIGHEST,
        preferred_element_type=jnp.float32)


def _recon(u, v, bm=512, bn=512):
    n0 = u.shape[0]
    bm, bn = min(bm, n0), min(bn, n0)
    kk = u.shape[1]
    grid = (n0 // bm, n0 // bn)
    return pl.pallas_call(
        _recon_body,
        grid=grid,
        in_specs=[
            pl.BlockSpec((bm, kk), lambda i, j: (i, 0)),
            pl.BlockSpec((bn, kk), lambda i, j: (j, 0)),
        ],
        out_specs=pl.BlockSpec((bm, bn), lambda i, j: (i, j)),
        out_shape=jax.ShapeDtypeStruct((n0, n0), jnp.float32),
    )(u, v)


def _sc_topk(bmax_flat, gre2, gim2, nblocks, nkeep):
    ns = 16
    pw = nblocks // ns
    nv1 = pw // 16
    rpw = nkeep // ns
    mesh = plsc.VectorSubcoreMesh(core_axis_name="c", subcore_axis_name="s")

    def body(bmax_hbm, gre_hbm, gim_hbm, a_out, b_out, i_out,
             vals1, gidx1, res_v, res_i, merge_v, merge_i, blk_v, blk_i,
             grer, gimr, mvals, mgidx, fin_v, fin_p, outa, outb, outi,
             sh_v, sh_i, sem):
        sid = lax.axis_index("s")
        cid = lax.axis_index("c")
        lane = lax.iota(jnp.int32, 16)
        m0 = lane == 0
        neg = jnp.full((16,), -jnp.inf, jnp.float32)

        def topk_scan(vals_ref, gidx_ref, nv, out_v_ref, out_i_ref):
            def one_pass(p, _):
                def scan4(i, carry):
                    bv, bi = carry
                    for u in range(4):
                        off = (i * 4 + u) * 16
                        x = vals_ref[pl.ds(off, 16)]
                        take = x > bv
                        bv = jnp.where(take, x, bv)
                        bi = jnp.where(take, off + lane, bi)
                    return bv, bi
                bv, bi = lax.fori_loop(0, nv // 4, scan4,
                                       (neg, jnp.zeros((16,), jnp.int32)))
                dnums = lax.GatherDimensionNumbers(
                    offset_dims=(), collapsed_slice_dims=(0,),
                    start_index_map=(0,))

                def _perm(x, pm):
                    return lax.gather(
                        x, pm[:, None], dnums, slice_sizes=(1,),
                        mode=lax.GatherScatterMode.PROMISE_IN_BOUNDS)

                for s in (8, 4, 2, 1):
                    perm = (lane + s) & 15
                    vs = _perm(bv, perm)
                    is_ = _perm(bi, perm)
                    take = vs > bv
                    bv = jnp.where(take, vs, bv)
                    bi = jnp.where(take, is_, bi)
                gv = plsc.load_gather(gidx_ref, [bi])
                pos = jnp.zeros((16,), jnp.int32) + p
                plsc.store_scatter(out_v_ref, [pos], bv, mask=m0)
                plsc.store_scatter(out_i_ref, [pos], gv, mask=m0)
                plsc.store_scatter(vals_ref, [bi], neg, mask=m0)
                return 0
            lax.fori_loop(0, nkeep, one_pass, 0)

        base = sid * pw
        pltpu.sync_copy(bmax_hbm.at[pl.ds(base, pw)], vals1)

        def fill(i, _):
            gidx1[pl.ds(i * 16, 16)] = base + i * 16 + lane
            return 0
        lax.fori_loop(0, nv1, fill, 0)
        topk_scan(vals1, gidx1, nv1, res_v, res_i)

        pltpu.sync_copy(res_v, sh_v.at[pl.ds(sid * nkeep, nkeep)])
        pltpu.sync_copy(res_i, sh_i.at[pl.ds(sid * nkeep, nkeep)])
        plsc.subcore_barrier()
        pltpu.sync_copy(sh_v, merge_v)
        pltpu.sync_copy(sh_i, merge_i)
        topk_scan(merge_v, merge_i, (ns * nkeep) // 16, blk_v, blk_i)

        pltpu.async_copy(gre_hbm.at[blk_i], grer, sem).wait()
        pltpu.async_copy(gim_hbm.at[blk_i], gimr, sem).wait()
        for t in range(rpw):
            rowv = jnp.zeros((16,), jnp.int32) + (sid * rpw + t)
            for o in range(8):
                col = o * 16 + lane
                rv = plsc.load_gather(grer, [rowv, col])
                iv = plsc.load_gather(gimr, [rowv, col])
                mvals[pl.ds((t * 8 + o) * 16, 16)] = rv * rv + iv * iv
                mgidx[pl.ds((t * 8 + o) * 16, 16)] = rowv * 128 + col
        topk_scan(mvals, mgidx, rpw * 8, res_v, res_i)

        plsc.subcore_barrier()
        pltpu.sync_copy(res_v, sh_v.at[pl.ds(sid * nkeep, nkeep)])
        pltpu.sync_copy(res_i, sh_i.at[pl.ds(sid * nkeep, nkeep)])
        plsc.subcore_barrier()
        pltpu.sync_copy(sh_v, merge_v)
        pltpu.sync_copy(sh_i, merge_i)
        topk_scan(merge_v, merge_i, (ns * nkeep) // 16, fin_v, fin_p)

        @pl.when((sid == 0) & (cid == 0))
        def _():
            for g in range(nkeep // 16):
                pv = fin_p[pl.ds(g * 16, 16)]
                rowv = pv >> 7
                offv = pv & 127
                outa[pl.ds(g * 16, 16)] = plsc.load_gather(grer, [rowv, offv])
                outb[pl.ds(g * 16, 16)] = plsc.load_gather(gimr, [rowv, offv])
                outi[pl.ds(g * 16, 16)] = (
                    plsc.load_gather(blk_i, [rowv]) * 128 + offv)
            pltpu.sync_copy(outa, a_out)
            pltpu.sync_copy(outb, b_out)
            pltpu.sync_copy(outi, i_out)

    run = functools.partial(
        pl.kernel,
        mesh=mesh,
        compiler_params=pltpu.CompilerParams(needs_layout_passes=False),
        out_type=[
            jax.ShapeDtypeStruct((nkeep,), jnp.float32),
            jax.ShapeDtypeStruct((nkeep,), jnp.float32),
            jax.ShapeDtypeStruct((nkeep,), jnp.int32),
        ],
        scratch_types=[
            pltpu.VMEM((pw,), jnp.float32),
            pltpu.VMEM((pw,), jnp.int32),
            pltpu.VMEM((nkeep,), jnp.float32),
            pltpu.VMEM((nkeep,), jnp.int32),
            pltpu.VMEM((ns * nkeep,), jnp.float32),
            pltpu.VMEM((ns * nkeep,), jnp.int32),
            pltpu.VMEM((nkeep,), jnp.float32),
            pltpu.VMEM((nkeep,), jnp.int32),
            pltpu.VMEM((nkeep, 128), jnp.float32),
            pltpu.VMEM((nkeep, 128), jnp.float32),
            pltpu.VMEM((rpw * 128,), jnp.float32),
            pltpu.VMEM((rpw * 128,), jnp.int32),
            pltpu.VMEM((nkeep,), jnp.float32),
            pltpu.VMEM((nkeep,), jnp.int32),
            pltpu.VMEM((nkeep,), jnp.float32),
            pltpu.VMEM((nkeep,), jnp.float32),
            pltpu.VMEM((nkeep,), jnp.int32),
            pltpu.VMEM_SHARED((ns * nkeep,), jnp.float32),
            pltpu.VMEM_SHARED((ns * nkeep,), jnp.int32),
            pltpu.SemaphoreType.DMA,
        ],
    )(body)
    return run(bmax_flat, gre2, gim2)


N_KEEP = 64


def kernel(weight_matrix):
    n0, n1 = weight_matrix.shape
    f1 = int(round(np.sqrt(n1)))
    ksl = n1 // 2 // f1 + 1
    align = 2 * LANE if n1 > 2048 else LANE
    while (f1 * ksl) % align:
        ksl += 1
    ncp = f1 * ksl
    c0, s0 = _gather_tables(n0)
    c0 = jnp.asarray(c0)
    s0 = jnp.asarray(s0)

    cs_np = np.arange(ncp)
    truec_np = f1 * (cs_np % ksl) + cs_np // ksl
    mask = jnp.asarray((truec_np <= n1 // 2)
                       .astype(np.float32).reshape(1, ncp))

    wt = _transpose(weight_matrix)
    z1re, z1im, _ = _rowfft_ct(wt, ksl)
    f2re = _transpose(z1re)
    f2im = _transpose(z1im)
    gre, gim, bmax, f0 = _colfft_ct(f2re, f2im, mask)

    nblocks = n0 * (ncp // LANE)
    a, b, idx = _sc_topk(bmax.reshape(-1),
                         gre.reshape(nblocks, LANE),
                         gim.reshape(nblocks, LANE),
                         nblocks, N_KEEP)
    rp = idx // ncp
    r = (rp % f0) * f0 + rp // f0
    cp = idx % ncp
    c = f1 * (cp % ksl) + cp // ksl

    u, v = _basis(c0, s0, a.reshape(1, -1), b.reshape(1, -1),
                  r.reshape(1, -1), c.reshape(1, -1))
    return _recon(u, v)
